# baseline (speedup 1.0000x reference)
import functools

import jax
import jax.numpy as jnp
from jax import lax
from jax.experimental import pallas as pl
from jax.experimental.pallas import tpu as pltpu
from jax.experimental.pallas import tpu_sc as plsc

B, N, E, D = 4, 2048, 32768, 128
H1, H2 = 256, 128
NC, NS = 2, 16
NW = NC * NS
GC = 128
SC_CHUNK = 32
BE = 512
F = B * D


def _relu(x):
    return jnp.maximum(x, 0.0)


def _mm(x, w):
    return jax.lax.dot_general(x, w, (((x.ndim - 1,), (0,)), ((), ())),
                               preferred_element_type=jnp.float32)


def _prenet_body(x_ref, w1_ref, b1_ref, w2_ref, b2_ref, o_ref):
    x = x_ref[0].astype(jnp.bfloat16)
    h = _relu(_mm(x, w1_ref[...]) + b1_ref[...]).astype(jnp.bfloat16)
    o_ref[0] = _relu(_mm(h, w2_ref[...]) + b2_ref[...]).astype(jnp.bfloat16)


def _run_prenet(x, wb):
    (w1, b1), (w2, b2) = wb
    g, r, d = x.shape
    return pl.pallas_call(
        _prenet_body,
        grid=(g,),
        in_specs=[
            pl.BlockSpec((1, r, d), lambda i: (i, 0, 0)),
            pl.BlockSpec((d, H1), lambda i: (0, 0)),
            pl.BlockSpec((1, H1), lambda i: (0, 0)),
            pl.BlockSpec((H1, H2), lambda i: (0, 0)),
            pl.BlockSpec((1, H2), lambda i: (0, 0)),
        ],
        out_specs=pl.BlockSpec((1, r, H2), lambda i: (i, 0, 0)),
        out_shape=jax.ShapeDtypeStruct((g, r, H2), jnp.bfloat16),
    )(x, w1.astype(jnp.bfloat16), b1.reshape(1, H1),
      w2.astype(jnp.bfloat16), b2.reshape(1, H2))


def _sites_prenet_body(x_ref, w1_ref, b1_ref, w2_ref, b2_ref,
                       tab_ref, s1_ref):
    ys = []
    for b in range(B):
        x = x_ref[b].astype(jnp.bfloat16)
        h = _relu(_mm(x, w1_ref[...]) + b1_ref[...]).astype(jnp.bfloat16)
        y = _relu(_mm(h, w2_ref[...]) + b2_ref[...]).astype(jnp.bfloat16)
        s1_ref[b] = y
        ys.append(y)
    for b in range(2):
        hi = jax.lax.bitcast_convert_type(ys[b], jnp.uint16).astype(jnp.uint32)
        lo = jax.lax.bitcast_convert_type(ys[b + 2], jnp.uint16).astype(jnp.uint32)
        w = (hi << 16) | lo
        tab_ref[:, b * D:(b + 1) * D] = jax.lax.bitcast_convert_type(w, jnp.int32)


def _run_sites_prenet(sites, wb):
    (w1, b1), (w2, b2) = wb
    return pl.pallas_call(
        _sites_prenet_body,
        in_specs=[
            pl.BlockSpec((B, N, D), lambda: (0, 0, 0)),
            pl.BlockSpec((D, H1), lambda: (0, 0)),
            pl.BlockSpec((1, H1), lambda: (0, 0)),
            pl.BlockSpec((H1, H2), lambda: (0, 0)),
            pl.BlockSpec((1, H2), lambda: (0, 0)),
        ],
        out_specs=[
            pl.BlockSpec((N, F // 2), lambda: (0, 0)),
            pl.BlockSpec((B, N, H2), lambda: (0, 0, 0)),
        ],
        out_shape=[
            jax.ShapeDtypeStruct((N, F // 2), jnp.int32),
            jax.ShapeDtypeStruct((B, N, H2), jnp.bfloat16),
        ],
    )(sites, w1.astype(jnp.bfloat16), b1.reshape(1, H1),
      w2.astype(jnp.bfloat16), b2.reshape(1, H2))


def _sc_gather_body(nrows, table_hbm, idx_hbm, out_hbm, idx_v, rows_v, sem):
    wid = lax.axis_index("s") * NC + lax.axis_index("c")
    rows_per_w = nrows // NW
    base = wid * rows_per_w
    for k in range(rows_per_w // GC):
        start = base + k * GC
        pltpu.sync_copy(idx_hbm.at[pl.ds(start, GC)], idx_v)
        pltpu.async_copy(table_hbm.at[idx_v], rows_v, sem).wait()
        pltpu.sync_copy(rows_v, out_hbm.at[pl.ds(start, GC)])


def _sc_gather(table, idx_cat):
    nrows = idx_cat.shape[0]
    mesh = plsc.VectorSubcoreMesh(core_axis_name="c", subcore_axis_name="s",
                                  num_cores=NC, num_subcores=NS)
    fn = pl.kernel(
        functools.partial(_sc_gather_body, nrows),
        out_type=jax.ShapeDtypeStruct((nrows, F // 2), jnp.int32),
        mesh=mesh,
        scratch_types=[
            pltpu.VMEM((GC,), jnp.int32),
            pltpu.VMEM((GC, F // 2), jnp.int32),
            pltpu.SemaphoreType.DMA,
        ],
    )
    return fn(table, idx_cat)


def _edge_body(bonds_ref, s1_ref, s2_ref, st1_ref,
               wb1_ref, bb1_ref, wb2_ref, bb2_ref,
               wm1_ref, bm1_ref, wm2_ref, bm2_ref, wm3_ref, bm3_ref,
               outb_ref, b2t_ref, esum_ref):
    parts = []
    for b in range(B):
        x = bonds_ref[b]
        xb = x.astype(jnp.bfloat16)
        h = _relu(_mm(xb, wb1_ref[...]) + bb1_ref[...]).astype(jnp.bfloat16)
        bonds1 = _relu(_mm(h, wb2_ref[...]) + bb2_ref[...]).astype(jnp.bfloat16)
        cols = pl.ds((b % 2) * D, D)
        if b < 2:
            mask = jnp.int32(-65536)
            s1 = jax.lax.bitcast_convert_type(s1_ref[:, cols] & mask, jnp.float32)
            s2 = jax.lax.bitcast_convert_type(s2_ref[:, cols] & mask, jnp.float32)
        else:
            s1 = jax.lax.bitcast_convert_type(s1_ref[:, cols] << 16, jnp.float32)
            s2 = jax.lax.bitcast_convert_type(s2_ref[:, cols] << 16, jnp.float32)
        s1 = s1.astype(jnp.bfloat16)
        s2 = s2.astype(jnp.bfloat16)
        sconst = _mm(st1_ref[b:b + 1, :], wm1_ref[3 * H2:4 * H2, :])
        t = (_mm(s1, wm1_ref[0:H2, :]) + _mm(s2, wm1_ref[H2:2 * H2, :])
             + _mm(bonds1, wm1_ref[2 * H2:3 * H2, :]) + sconst + bm1_ref[...])
        t = _relu(t).astype(jnp.bfloat16)
        t = _relu(_mm(t, wm2_ref[...]) + bm2_ref[...]).astype(jnp.bfloat16)
        b2 = _mm(t, wm3_ref[...]) + bm3_ref[...]
        outb_ref[b] = x + b2
        b2t_ref[:, b, :] = b2
        parts.append(jnp.sum(b2, axis=0, keepdims=True))
    b2t_ref[:, B, :] = jnp.ones((BE, D), jnp.float32)
    parts.append(jnp.zeros((8 - B, H2), jnp.float32))
    psum = jnp.concatenate(parts, axis=0)

    @pl.when(pl.program_id(0) == 0)
    def _init():
        esum_ref[...] = psum

    @pl.when(pl.program_id(0) != 0)
    def _acc():
        esum_ref[...] = esum_ref[...] + psum


def _run_edge(bonds, g, states1_pad, params, off, ne):
    (wb1, bb1), (wb2, bb2) = params['bonds_fc']
    (wm1, bm1), (wm2, bm2), (wm3, bm3) = params['bond_mlp']
    nblk = ne // BE
    oblk = off // BE
    return pl.pallas_call(
        _edge_body,
        grid=(nblk,),
        in_specs=[
            pl.BlockSpec((B, BE, D), lambda e: (0, e + oblk, 0)),
            pl.BlockSpec((BE, F // 2), lambda e: (e, 0)),
            pl.BlockSpec((BE, F // 2), lambda e: (e + nblk, 0)),
            pl.BlockSpec((8, D), lambda e: (0, 0)),
            pl.BlockSpec((D, H1), lambda e: (0, 0)),
            pl.BlockSpec((1, H1), lambda e: (0, 0)),
            pl.BlockSpec((H1, H2), lambda e: (0, 0)),
            pl.BlockSpec((1, H2), lambda e: (0, 0)),
            pl.BlockSpec((4 * H2, 256), lambda e: (0, 0)),
            pl.BlockSpec((1, 256), lambda e: (0, 0)),
            pl.BlockSpec((256, 256), lambda e: (0, 0)),
            pl.BlockSpec((1, 256), lambda e: (0, 0)),
            pl.BlockSpec((256, H2), lambda e: (0, 0)),
            pl.BlockSpec((1, H2), lambda e: (0, 0)),
        ],
        out_specs=[
            pl.BlockSpec((B, BE, D), lambda e: (0, e, 0)),
            pl.BlockSpec((BE, B + 1, D), lambda e: (e, 0, 0)),
            pl.BlockSpec((8, H2), lambda e: (0, 0)),
        ],
        out_shape=[
            jax.ShapeDtypeStruct((B, ne, D), jnp.float32),
            jax.ShapeDtypeStruct((ne, B + 1, D), jnp.float32),
            jax.ShapeDtypeStruct((8, H2), jnp.float32),
        ],
    )(bonds, g, g, states1_pad,
      wb1.astype(jnp.bfloat16), bb1.reshape(1, H1),
      wb2.astype(jnp.bfloat16), bb2.reshape(1, H2),
      wm1.astype(jnp.bfloat16), bm1.reshape(1, 256),
      wm2.astype(jnp.bfloat16), bm2.reshape(1, 256),
      wm3.astype(jnp.bfloat16), bm3.reshape(1, H2))


FS = (B + 1) * D


def _sc_scatter_body(ne, b2t_hbm, idx_hbm, zrow_hbm, pool_hbm,
                     rows_v, idx_v, lsem, isem, ssem):
    cid = lax.axis_index("c")
    sid = lax.axis_index("s")
    wid = sid * NC + cid
    zrows = N // NS
    r0 = sid * zrows
    pltpu.sync_copy(zrow_hbm.at[pl.ds(r0, zrows)], pool_hbm.at[cid, pl.ds(r0, zrows)])
    plsc.subcore_barrier()
    pool_c = pool_hbm.at[cid]
    e_per_w = ne // NW
    base = wid * e_per_w
    nch = e_per_w // SC_CHUNK
    nb = 4
    loads = [None] * nb
    scats = [[] for _ in range(nb)]

    def start_load(k):
        bi = k % nb
        st = base + k * SC_CHUNK
        loads[bi] = (
            pltpu.async_copy(b2t_hbm.at[pl.ds(st, SC_CHUNK)], rows_v.at[bi],
                             lsem),
            pltpu.async_copy(idx_hbm.at[pl.ds(st, SC_CHUNK)], idx_v.at[bi],
                             isem),
        )

    start_load(0)
    start_load(1)
    for k in range(nch):
        bi = k % nb
        for d in loads[bi]:
            d.wait()
        for j in range(SC_CHUNK // 16):
            idx_vec = idx_v[bi, pl.ds(j * 16, 16)]
            scats[bi].append(
                pltpu.async_copy(rows_v.at[bi, pl.ds(j * 16, 16)],
                                 pool_c.at[idx_vec], ssem.at[bi], add=True))
        if k + 2 < nch:
            nbi = (k + 2) % nb
            for d in scats[nbi]:
                d.wait()
            scats[nbi] = []
            start_load(k + 2)
    for bl in scats:
        for d in bl:
            d.wait()


def _sc_scatter(b2t, idx1):
    ne = idx1.shape[0]
    mesh = plsc.VectorSubcoreMesh(core_axis_name="c", subcore_axis_name="s",
                                  num_cores=NC, num_subcores=NS)
    fn = pl.kernel(
        functools.partial(_sc_scatter_body, ne),
        out_type=jax.ShapeDtypeStruct((NC, N, FS), jnp.float32),
        mesh=mesh,
        scratch_types=[
            pltpu.VMEM((4, SC_CHUNK, FS), jnp.float32),
            pltpu.VMEM((4, SC_CHUNK), jnp.int32),
            pltpu.SemaphoreType.DMA,
            pltpu.SemaphoreType.DMA,
            pltpu.SemaphoreType.DMA((4,)),
        ],
    )
    zrow = jnp.zeros((N, FS), jnp.float32)
    return fn(b2t, idx1, zrow)


def _site_body(pool_ref, poolb_ref, cnt_ref, cntb_ref, sites1_ref, sites_ref, st1row_ref,
               st1_ref, stpad_ref, esum_ref,
               ws1_ref, bs1_ref, ws2_ref, bs2_ref, ws3_ref, bs3_ref,
               wt1_ref, bt1_ref, wt2_ref, bt2_ref, wt3_ref, bt3_ref,
               osites_ref, ostates_ref, smean_ref):
    b = pl.program_id(0)
    psum = (pool_ref[0, :, 0, 0, :] + pool_ref[1, :, 0, 0, :]
            + poolb_ref[0, :, 0, 0, :] + poolb_ref[1, :, 0, 0, :])
    c = (cnt_ref[0, :, 0, 0, 0:1] + cnt_ref[1, :, 0, 0, 0:1]
         + cntb_ref[0, :, 0, 0, 0:1] + cntb_ref[1, :, 0, 0, 0:1])
    pool = (psum / jnp.maximum(c, 1.0)).astype(jnp.bfloat16)
    s1b = sites1_ref[0]
    sconst = _mm(st1row_ref[0, 0:1, :], ws1_ref[2 * H2:3 * H2, :])
    t = _relu(_mm(pool, ws1_ref[0:H2, :]) + _mm(s1b, ws1_ref[H2:2 * H2, :])
              + sconst + bs1_ref[...]).astype(jnp.bfloat16)
    t = _relu(_mm(t, ws2_ref[...]) + bs2_ref[...]).astype(jnp.bfloat16)
    s2out = _relu(_mm(t, ws3_ref[...]) + bs3_ref[...])
    osites_ref[0] = sites_ref[0] + s2out

    mean_row = jnp.sum(s2out, axis=0, keepdims=True) / float(N)
    rows = lax.broadcasted_iota(jnp.int32, (8, H2), 0)
    contrib = jnp.where(rows == b, jnp.broadcast_to(mean_row, (8, H2)), 0.0)

    @pl.when(b == 0)
    def _init():
        smean_ref[...] = contrib

    @pl.when(b != 0)
    def _acc():
        smean_ref[...] = smean_ref[...] + contrib

    @pl.when(b == B - 1)
    def _states():
        bmean = (esum_ref[...] / float(E)).astype(jnp.bfloat16)
        v = (_mm(bmean, wt1_ref[0:H2, :])
             + _mm(smean_ref[...].astype(jnp.bfloat16), wt1_ref[H2:2 * H2, :])
             + _mm(st1_ref[...], wt1_ref[2 * H2:3 * H2, :]) + bt1_ref[...])
        v = _relu(v).astype(jnp.bfloat16)
        v = _relu(_mm(v, wt2_ref[...]) + bt2_ref[...]).astype(jnp.bfloat16)
        v = _relu(_mm(v, wt3_ref[...]) + bt3_ref[...])
        ostates_ref[...] = stpad_ref[...] + v


def _run_site(poola, poolb, sites1, sites, states1_pad, states_pad, esum, params):
    (ws1, bs1), (ws2, bs2), (ws3, bs3) = params['site_mlp']
    (wt1, bt1), (wt2, bt2), (wt3, bt3) = params['state_mlp']
    pool5 = poola.reshape(NC, N, B + 1, 1, D)
    pool5b = poolb.reshape(NC, N, B + 1, 1, D)
    st1rows = states1_pad.reshape(8, 1, D)
    return pl.pallas_call(
        _site_body,
        grid=(B,),
        in_specs=[
            pl.BlockSpec((NC, N, 1, 1, D), lambda b: (0, 0, b, 0, 0)),
            pl.BlockSpec((NC, N, 1, 1, D), lambda b: (0, 0, b, 0, 0)),
            pl.BlockSpec((NC, N, 1, 1, D), lambda b: (0, 0, B, 0, 0)),
            pl.BlockSpec((NC, N, 1, 1, D), lambda b: (0, 0, B, 0, 0)),
            pl.BlockSpec((1, N, D), lambda b: (b, 0, 0)),
            pl.BlockSpec((1, N, D), lambda b: (b, 0, 0)),
            pl.BlockSpec((1, 1, D), lambda b: (b, 0, 0)),
            pl.BlockSpec((8, D), lambda b: (0, 0)),
            pl.BlockSpec((8, D), lambda b: (0, 0)),
            pl.BlockSpec((8, H2), lambda b: (0, 0)),
            pl.BlockSpec((3 * H2, 256), lambda b: (0, 0)),
            pl.BlockSpec((1, 256), lambda b: (0, 0)),
            pl.BlockSpec((256, 256), lambda b: (0, 0)),
            pl.BlockSpec((1, 256), lambda b: (0, 0)),
            pl.BlockSpec((256, H2), lambda b: (0, 0)),
            pl.BlockSpec((1, H2), lambda b: (0, 0)),
            pl.BlockSpec((3 * H2, 256), lambda b: (0, 0)),
            pl.BlockSpec((1, 256), lambda b: (0, 0)),
            pl.BlockSpec((256, 256), lambda b: (0, 0)),
            pl.BlockSpec((1, 256), lambda b: (0, 0)),
            pl.BlockSpec((256, H2), lambda b: (0, 0)),
            pl.BlockSpec((1, H2), lambda b: (0, 0)),
        ],
        out_specs=[
            pl.BlockSpec((1, N, D), lambda b: (b, 0, 0)),
            pl.BlockSpec((8, D), lambda b: (0, 0)),
        ],
        out_shape=[
            jax.ShapeDtypeStruct((B, N, D), jnp.float32),
            jax.ShapeDtypeStruct((8, D), jnp.float32),
        ],
        scratch_shapes=[pltpu.VMEM((8, H2), jnp.float32)],
    )(pool5, pool5b, pool5, pool5b, sites1, sites, st1rows, states1_pad, states_pad, esum,
      ws1.astype(jnp.bfloat16), bs1.reshape(1, 256),
      ws2.astype(jnp.bfloat16), bs2.reshape(1, 256),
      ws3.astype(jnp.bfloat16), bs3.reshape(1, H2),
      wt1.astype(jnp.bfloat16), bt1.reshape(1, 256),
      wt2.astype(jnp.bfloat16), bt2.reshape(1, 256),
      wt3.astype(jnp.bfloat16), bt3.reshape(1, H2))


def kernel(sites, bonds, states, indices1, indices2, params):
    idx1 = indices1.astype(jnp.int32)
    idx2 = indices2.astype(jnp.int32)
    states_pad = jnp.pad(states, ((0, 8 - B), (0, 0)))

    states1_pad = _run_prenet(states_pad.reshape(8, 1, D),
                              params['states_fc']).reshape(8, H2)
    table, sites1 = _run_sites_prenet(sites, params['sites_fc'])

    EH = E // 2
    idx1a, idx1b = idx1[:EH], idx1[EH:]
    idx2a, idx2b = idx2[:EH], idx2[EH:]
    ga = _sc_gather(table, jnp.concatenate([idx1a, idx2a]))
    gb = _sc_gather(table, jnp.concatenate([idx1b, idx2b]))

    bonds_a, b2t_a, esum_a = _run_edge(bonds, ga, states1_pad, params, 0, EH)
    bonds_b, b2t_b, esum_b = _run_edge(bonds, gb, states1_pad, params, EH, EH)

    poola = _sc_scatter(b2t_a.reshape(EH, FS), idx1a)
    poolb = _sc_scatter(b2t_b.reshape(EH, FS), idx1b)

    bonds_out = jnp.concatenate([bonds_a, bonds_b], axis=1)
    sites_out, states_out_pad = _run_site(poola, poolb, sites1, sites,
                                          states1_pad, states_pad,
                                          esum_a + esum_b, params)
    return sites_out, bonds_out, states_out_pad[:B]


# speedup vs baseline: 1.4935x; 1.4935x over previous
import functools

import jax
import jax.numpy as jnp
from jax import lax
from jax.experimental import pallas as pl
from jax.experimental.pallas import tpu as pltpu
from jax.experimental.pallas import tpu_sc as plsc

B, N, E, D = 4, 2048, 32768, 128
H1, H2 = 256, 128
NC, NS = 2, 16
NW = NC * NS
GC = 128
SC_CHUNK = 32
BE = 512
F = B * D


def _relu(x):
    return jnp.maximum(x, 0.0)


def _mm(x, w):
    return jax.lax.dot_general(x, w, (((x.ndim - 1,), (0,)), ((), ())),
                               preferred_element_type=jnp.float32)


def _prenet_body(x_ref, w1_ref, b1_ref, w2_ref, b2_ref, o_ref):
    x = x_ref[0].astype(jnp.bfloat16)
    h = _relu(_mm(x, w1_ref[...]) + b1_ref[...]).astype(jnp.bfloat16)
    o_ref[0] = _relu(_mm(h, w2_ref[...]) + b2_ref[...]).astype(jnp.bfloat16)


def _run_prenet(x, wb):
    (w1, b1), (w2, b2) = wb
    g, r, d = x.shape
    return pl.pallas_call(
        _prenet_body,
        grid=(g,),
        in_specs=[
            pl.BlockSpec((1, r, d), lambda i: (i, 0, 0)),
            pl.BlockSpec((d, H1), lambda i: (0, 0)),
            pl.BlockSpec((1, H1), lambda i: (0, 0)),
            pl.BlockSpec((H1, H2), lambda i: (0, 0)),
            pl.BlockSpec((1, H2), lambda i: (0, 0)),
        ],
        out_specs=pl.BlockSpec((1, r, H2), lambda i: (i, 0, 0)),
        out_shape=jax.ShapeDtypeStruct((g, r, H2), jnp.bfloat16),
    )(x, w1.astype(jnp.bfloat16), b1.reshape(1, H1),
      w2.astype(jnp.bfloat16), b2.reshape(1, H2))


def _sites_prenet_body(x_ref, w1_ref, b1_ref, w2_ref, b2_ref,
                       tab_ref, s1_ref):
    ys = []
    for b in range(B):
        x = x_ref[b].astype(jnp.bfloat16)
        h = _relu(_mm(x, w1_ref[...]) + b1_ref[...]).astype(jnp.bfloat16)
        y = _relu(_mm(h, w2_ref[...]) + b2_ref[...]).astype(jnp.bfloat16)
        s1_ref[b] = y
        ys.append(y)
    for b in range(2):
        hi = jax.lax.bitcast_convert_type(ys[b], jnp.uint16).astype(jnp.uint32)
        lo = jax.lax.bitcast_convert_type(ys[b + 2], jnp.uint16).astype(jnp.uint32)
        w = (hi << 16) | lo
        tab_ref[:, b * D:(b + 1) * D] = jax.lax.bitcast_convert_type(w, jnp.int32)


def _run_sites_prenet(sites, wb):
    (w1, b1), (w2, b2) = wb
    return pl.pallas_call(
        _sites_prenet_body,
        in_specs=[
            pl.BlockSpec((B, N, D), lambda: (0, 0, 0)),
            pl.BlockSpec((D, H1), lambda: (0, 0)),
            pl.BlockSpec((1, H1), lambda: (0, 0)),
            pl.BlockSpec((H1, H2), lambda: (0, 0)),
            pl.BlockSpec((1, H2), lambda: (0, 0)),
        ],
        out_specs=[
            pl.BlockSpec((N, F // 2), lambda: (0, 0)),
            pl.BlockSpec((B, N, H2), lambda: (0, 0, 0)),
        ],
        out_shape=[
            jax.ShapeDtypeStruct((N, F // 2), jnp.int32),
            jax.ShapeDtypeStruct((B, N, H2), jnp.bfloat16),
        ],
    )(sites, w1.astype(jnp.bfloat16), b1.reshape(1, H1),
      w2.astype(jnp.bfloat16), b2.reshape(1, H2))


def _sc_gather_body(nrows, table_hbm, idx_hbm, out_hbm, idx_v, rows_v, sem):
    wid = lax.axis_index("s") * NC + lax.axis_index("c")
    rows_per_w = nrows // NW
    base = wid * rows_per_w
    for k in range(rows_per_w // GC):
        start = base + k * GC
        pltpu.sync_copy(idx_hbm.at[pl.ds(start, GC)], idx_v)
        pltpu.async_copy(table_hbm.at[idx_v], rows_v, sem).wait()
        pltpu.sync_copy(rows_v, out_hbm.at[pl.ds(start, GC)])


def _sc_gather(table, idx_cat):
    nrows = idx_cat.shape[0]
    mesh = plsc.VectorSubcoreMesh(core_axis_name="c", subcore_axis_name="s",
                                  num_cores=NC, num_subcores=NS)
    fn = pl.kernel(
        functools.partial(_sc_gather_body, nrows),
        out_type=jax.ShapeDtypeStruct((nrows, F // 2), jnp.int32),
        mesh=mesh,
        scratch_types=[
            pltpu.VMEM((GC,), jnp.int32),
            pltpu.VMEM((GC, F // 2), jnp.int32),
            pltpu.SemaphoreType.DMA,
        ],
    )
    return fn(table, idx_cat)


def _edge_body(bonds_ref, s1_ref, s2_ref, st1_ref,
               wb1_ref, bb1_ref, wb2_ref, bb2_ref,
               wm1_ref, bm1_ref, wm2_ref, bm2_ref, wm3_ref, bm3_ref,
               outb_ref, b2t_ref, esum_ref):
    parts = []
    for b in range(B):
        x = bonds_ref[b]
        xb = x.astype(jnp.bfloat16)
        h = _relu(_mm(xb, wb1_ref[...]) + bb1_ref[...]).astype(jnp.bfloat16)
        bonds1 = _relu(_mm(h, wb2_ref[...]) + bb2_ref[...]).astype(jnp.bfloat16)
        cols = pl.ds((b % 2) * D, D)
        if b < 2:
            mask = jnp.int32(-65536)
            s1 = jax.lax.bitcast_convert_type(s1_ref[:, cols] & mask, jnp.float32)
            s2 = jax.lax.bitcast_convert_type(s2_ref[:, cols] & mask, jnp.float32)
        else:
            s1 = jax.lax.bitcast_convert_type(s1_ref[:, cols] << 16, jnp.float32)
            s2 = jax.lax.bitcast_convert_type(s2_ref[:, cols] << 16, jnp.float32)
        s1 = s1.astype(jnp.bfloat16)
        s2 = s2.astype(jnp.bfloat16)
        sconst = _mm(st1_ref[b:b + 1, :], wm1_ref[3 * H2:4 * H2, :])
        t = (_mm(s1, wm1_ref[0:H2, :]) + _mm(s2, wm1_ref[H2:2 * H2, :])
             + _mm(bonds1, wm1_ref[2 * H2:3 * H2, :]) + sconst + bm1_ref[...])
        t = _relu(t).astype(jnp.bfloat16)
        t = _relu(_mm(t, wm2_ref[...]) + bm2_ref[...]).astype(jnp.bfloat16)
        b2 = _mm(t, wm3_ref[...]) + bm3_ref[...]
        outb_ref[b] = x + b2
        b2t_ref[:, b, :] = b2
        parts.append(jnp.sum(b2, axis=0, keepdims=True))
    b2t_ref[:, B, :] = jnp.ones((BE, D), jnp.float32)
    parts.append(jnp.zeros((8 - B, H2), jnp.float32))
    psum = jnp.concatenate(parts, axis=0)

    @pl.when(pl.program_id(0) == 0)
    def _init():
        esum_ref[...] = psum

    @pl.when(pl.program_id(0) != 0)
    def _acc():
        esum_ref[...] = esum_ref[...] + psum


def _run_edge(bonds, g, states1_pad, params, off, ne):
    (wb1, bb1), (wb2, bb2) = params['bonds_fc']
    (wm1, bm1), (wm2, bm2), (wm3, bm3) = params['bond_mlp']
    nblk = ne // BE
    oblk = off // BE
    return pl.pallas_call(
        _edge_body,
        grid=(nblk,),
        in_specs=[
            pl.BlockSpec((B, BE, D), lambda e: (0, e + oblk, 0)),
            pl.BlockSpec((BE, F // 2), lambda e: (e, 0)),
            pl.BlockSpec((BE, F // 2), lambda e: (e + nblk, 0)),
            pl.BlockSpec((8, D), lambda e: (0, 0)),
            pl.BlockSpec((D, H1), lambda e: (0, 0)),
            pl.BlockSpec((1, H1), lambda e: (0, 0)),
            pl.BlockSpec((H1, H2), lambda e: (0, 0)),
            pl.BlockSpec((1, H2), lambda e: (0, 0)),
            pl.BlockSpec((4 * H2, 256), lambda e: (0, 0)),
            pl.BlockSpec((1, 256), lambda e: (0, 0)),
            pl.BlockSpec((256, 256), lambda e: (0, 0)),
            pl.BlockSpec((1, 256), lambda e: (0, 0)),
            pl.BlockSpec((256, H2), lambda e: (0, 0)),
            pl.BlockSpec((1, H2), lambda e: (0, 0)),
        ],
        out_specs=[
            pl.BlockSpec((B, BE, D), lambda e: (0, e, 0)),
            pl.BlockSpec((BE, B + 1, D), lambda e: (e, 0, 0)),
            pl.BlockSpec((8, H2), lambda e: (0, 0)),
        ],
        out_shape=[
            jax.ShapeDtypeStruct((B, ne, D), jnp.float32),
            jax.ShapeDtypeStruct((ne, B + 1, D), jnp.float32),
            jax.ShapeDtypeStruct((8, H2), jnp.float32),
        ],
    )(bonds, g, g, states1_pad,
      wb1.astype(jnp.bfloat16), bb1.reshape(1, H1),
      wb2.astype(jnp.bfloat16), bb2.reshape(1, H2),
      wm1.astype(jnp.bfloat16), bm1.reshape(1, 256),
      wm2.astype(jnp.bfloat16), bm2.reshape(1, 256),
      wm3.astype(jnp.bfloat16), bm3.reshape(1, H2))


FS = (B + 1) * D


def _sc_scatter_body(ne, b2t_hbm, idx_hbm, zrow_hbm, pool_hbm,
                     rows_v, idx_v, lsem, isem, ssem):
    cid = lax.axis_index("c")
    sid = lax.axis_index("s")
    wid = sid * NC + cid
    zrows = N // NS
    r0 = sid * zrows
    pltpu.sync_copy(zrow_hbm, rows_v.at[0])
    zds = [pltpu.async_copy(rows_v.at[0],
                            pool_hbm.at[cid, pl.ds(r0 + t * 32, 32)], lsem)
           for t in range(zrows // 32)]
    for d in zds:
        d.wait()
    plsc.subcore_barrier()
    pool_c = pool_hbm.at[cid]
    e_per_w = ne // NW
    base = wid * e_per_w
    nch = e_per_w // SC_CHUNK
    nb = 4
    loads = [None] * nb
    scats = [[] for _ in range(nb)]

    def start_load(k):
        bi = k % nb
        st = base + k * SC_CHUNK
        loads[bi] = (
            pltpu.async_copy(b2t_hbm.at[pl.ds(st, SC_CHUNK)], rows_v.at[bi],
                             lsem),
            pltpu.async_copy(idx_hbm.at[pl.ds(st, SC_CHUNK)], idx_v.at[bi],
                             isem),
        )

    start_load(0)
    start_load(1)
    for k in range(nch):
        bi = k % nb
        for d in loads[bi]:
            d.wait()
        for j in range(SC_CHUNK // 16):
            idx_vec = idx_v[bi, pl.ds(j * 16, 16)]
            scats[bi].append(
                pltpu.async_copy(rows_v.at[bi, pl.ds(j * 16, 16)],
                                 pool_c.at[idx_vec], ssem.at[bi], add=True))
        if k + 2 < nch:
            nbi = (k + 2) % nb
            for d in scats[nbi]:
                d.wait()
            scats[nbi] = []
            start_load(k + 2)
    for bl in scats:
        for d in bl:
            d.wait()


def _sc_scatter(b2t, idx1):
    ne = idx1.shape[0]
    mesh = plsc.VectorSubcoreMesh(core_axis_name="c", subcore_axis_name="s",
                                  num_cores=NC, num_subcores=NS)
    fn = pl.kernel(
        functools.partial(_sc_scatter_body, ne),
        out_type=jax.ShapeDtypeStruct((NC, N, FS), jnp.float32),
        mesh=mesh,
        scratch_types=[
            pltpu.VMEM((4, SC_CHUNK, FS), jnp.float32),
            pltpu.VMEM((4, SC_CHUNK), jnp.int32),
            pltpu.SemaphoreType.DMA,
            pltpu.SemaphoreType.DMA,
            pltpu.SemaphoreType.DMA((4,)),
        ],
    )
    zrow = jnp.zeros((32, FS), jnp.float32)
    return fn(b2t, idx1, zrow)


def _site_body(pool_ref, poolb_ref, cnt_ref, cntb_ref, sites1_ref, sites_ref, st1row_ref,
               st1_ref, stpad_ref, esum_ref,
               ws1_ref, bs1_ref, ws2_ref, bs2_ref, ws3_ref, bs3_ref,
               wt1_ref, bt1_ref, wt2_ref, bt2_ref, wt3_ref, bt3_ref,
               osites_ref, ostates_ref, smean_ref):
    b = pl.program_id(0)
    psum = (pool_ref[0, :, 0, 0, :] + pool_ref[1, :, 0, 0, :]
            + poolb_ref[0, :, 0, 0, :] + poolb_ref[1, :, 0, 0, :])
    c = (cnt_ref[0, :, 0, 0, 0:1] + cnt_ref[1, :, 0, 0, 0:1]
         + cntb_ref[0, :, 0, 0, 0:1] + cntb_ref[1, :, 0, 0, 0:1])
    pool = (psum / jnp.maximum(c, 1.0)).astype(jnp.bfloat16)
    s1b = sites1_ref[0]
    sconst = _mm(st1row_ref[0, 0:1, :], ws1_ref[2 * H2:3 * H2, :])
    t = _relu(_mm(pool, ws1_ref[0:H2, :]) + _mm(s1b, ws1_ref[H2:2 * H2, :])
              + sconst + bs1_ref[...]).astype(jnp.bfloat16)
    t = _relu(_mm(t, ws2_ref[...]) + bs2_ref[...]).astype(jnp.bfloat16)
    s2out = _relu(_mm(t, ws3_ref[...]) + bs3_ref[...])
    osites_ref[0] = sites_ref[0] + s2out

    mean_row = jnp.sum(s2out, axis=0, keepdims=True) / float(N)
    rows = lax.broadcasted_iota(jnp.int32, (8, H2), 0)
    contrib = jnp.where(rows == b, jnp.broadcast_to(mean_row, (8, H2)), 0.0)

    @pl.when(b == 0)
    def _init():
        smean_ref[...] = contrib

    @pl.when(b != 0)
    def _acc():
        smean_ref[...] = smean_ref[...] + contrib

    @pl.when(b == B - 1)
    def _states():
        bmean = (esum_ref[...] / float(E)).astype(jnp.bfloat16)
        v = (_mm(bmean, wt1_ref[0:H2, :])
             + _mm(smean_ref[...].astype(jnp.bfloat16), wt1_ref[H2:2 * H2, :])
             + _mm(st1_ref[...], wt1_ref[2 * H2:3 * H2, :]) + bt1_ref[...])
        v = _relu(v).astype(jnp.bfloat16)
        v = _relu(_mm(v, wt2_ref[...]) + bt2_ref[...]).astype(jnp.bfloat16)
        v = _relu(_mm(v, wt3_ref[...]) + bt3_ref[...])
        ostates_ref[...] = stpad_ref[...] + v


def _run_site(poola, poolb, sites1, sites, states1_pad, states_pad, esum, params):
    (ws1, bs1), (ws2, bs2), (ws3, bs3) = params['site_mlp']
    (wt1, bt1), (wt2, bt2), (wt3, bt3) = params['state_mlp']
    pool5 = poola.reshape(NC, N, B + 1, 1, D)
    pool5b = poolb.reshape(NC, N, B + 1, 1, D)
    st1rows = states1_pad.reshape(8, 1, D)
    return pl.pallas_call(
        _site_body,
        grid=(B,),
        in_specs=[
            pl.BlockSpec((NC, N, 1, 1, D), lambda b: (0, 0, b, 0, 0)),
            pl.BlockSpec((NC, N, 1, 1, D), lambda b: (0, 0, b, 0, 0)),
            pl.BlockSpec((NC, N, 1, 1, D), lambda b: (0, 0, B, 0, 0)),
            pl.BlockSpec((NC, N, 1, 1, D), lambda b: (0, 0, B, 0, 0)),
            pl.BlockSpec((1, N, D), lambda b: (b, 0, 0)),
            pl.BlockSpec((1, N, D), lambda b: (b, 0, 0)),
            pl.BlockSpec((1, 1, D), lambda b: (b, 0, 0)),
            pl.BlockSpec((8, D), lambda b: (0, 0)),
            pl.BlockSpec((8, D), lambda b: (0, 0)),
            pl.BlockSpec((8, H2), lambda b: (0, 0)),
            pl.BlockSpec((3 * H2, 256), lambda b: (0, 0)),
            pl.BlockSpec((1, 256), lambda b: (0, 0)),
            pl.BlockSpec((256, 256), lambda b: (0, 0)),
            pl.BlockSpec((1, 256), lambda b: (0, 0)),
            pl.BlockSpec((256, H2), lambda b: (0, 0)),
            pl.BlockSpec((1, H2), lambda b: (0, 0)),
            pl.BlockSpec((3 * H2, 256), lambda b: (0, 0)),
            pl.BlockSpec((1, 256), lambda b: (0, 0)),
            pl.BlockSpec((256, 256), lambda b: (0, 0)),
            pl.BlockSpec((1, 256), lambda b: (0, 0)),
            pl.BlockSpec((256, H2), lambda b: (0, 0)),
            pl.BlockSpec((1, H2), lambda b: (0, 0)),
        ],
        out_specs=[
            pl.BlockSpec((1, N, D), lambda b: (b, 0, 0)),
            pl.BlockSpec((8, D), lambda b: (0, 0)),
        ],
        out_shape=[
            jax.ShapeDtypeStruct((B, N, D), jnp.float32),
            jax.ShapeDtypeStruct((8, D), jnp.float32),
        ],
        scratch_shapes=[pltpu.VMEM((8, H2), jnp.float32)],
    )(pool5, pool5b, pool5, pool5b, sites1, sites, st1rows, states1_pad, states_pad, esum,
      ws1.astype(jnp.bfloat16), bs1.reshape(1, 256),
      ws2.astype(jnp.bfloat16), bs2.reshape(1, 256),
      ws3.astype(jnp.bfloat16), bs3.reshape(1, H2),
      wt1.astype(jnp.bfloat16), bt1.reshape(1, 256),
      wt2.astype(jnp.bfloat16), bt2.reshape(1, 256),
      wt3.astype(jnp.bfloat16), bt3.reshape(1, H2))


def kernel(sites, bonds, states, indices1, indices2, params):
    idx1 = indices1.astype(jnp.int32)
    idx2 = indices2.astype(jnp.int32)
    states_pad = jnp.pad(states, ((0, 8 - B), (0, 0)))

    states1_pad = _run_prenet(states_pad.reshape(8, 1, D),
                              params['states_fc']).reshape(8, H2)
    table, sites1 = _run_sites_prenet(sites, params['sites_fc'])

    EH = E // 2
    idx1a, idx1b = idx1[:EH], idx1[EH:]
    idx2a, idx2b = idx2[:EH], idx2[EH:]
    ga = _sc_gather(table, jnp.concatenate([idx1a, idx2a]))
    gb = _sc_gather(table, jnp.concatenate([idx1b, idx2b]))

    bonds_a, b2t_a, esum_a = _run_edge(bonds, ga, states1_pad, params, 0, EH)
    bonds_b, b2t_b, esum_b = _run_edge(bonds, gb, states1_pad, params, EH, EH)

    poola = _sc_scatter(b2t_a.reshape(EH, FS), idx1a)
    poolb = _sc_scatter(b2t_b.reshape(EH, FS), idx1b)

    bonds_out = jnp.concatenate([bonds_a, bonds_b], axis=1)
    sites_out, states_out_pad = _run_site(poola, poolb, sites1, sites,
                                          states1_pad, states_pad,
                                          esum_a + esum_b, params)
    return sites_out, bonds_out, states_out_pad[:B]


# speedup vs baseline: 1.5513x; 1.0387x over previous
import functools

import jax
import jax.numpy as jnp
from jax import lax
from jax.experimental import pallas as pl
from jax.experimental.pallas import tpu as pltpu
from jax.experimental.pallas import tpu_sc as plsc

B, N, E, D = 4, 2048, 32768, 128
H1, H2 = 256, 128
NC, NS = 2, 16
NW = NC * NS
GC = 128
SC_CHUNK = 32
BE = 512
F = B * D


def _relu(x):
    return jnp.maximum(x, 0.0)


def _mm(x, w):
    return jax.lax.dot_general(x, w, (((x.ndim - 1,), (0,)), ((), ())),
                               preferred_element_type=jnp.float32)


def _prenet_body(x_ref, w1_ref, b1_ref, w2_ref, b2_ref, o_ref):
    x = x_ref[0].astype(jnp.bfloat16)
    h = _relu(_mm(x, w1_ref[...]) + b1_ref[...]).astype(jnp.bfloat16)
    o_ref[0] = _relu(_mm(h, w2_ref[...]) + b2_ref[...]).astype(jnp.bfloat16)


def _run_prenet(x, wb):
    (w1, b1), (w2, b2) = wb
    g, r, d = x.shape
    return pl.pallas_call(
        _prenet_body,
        grid=(g,),
        in_specs=[
            pl.BlockSpec((1, r, d), lambda i: (i, 0, 0)),
            pl.BlockSpec((d, H1), lambda i: (0, 0)),
            pl.BlockSpec((1, H1), lambda i: (0, 0)),
            pl.BlockSpec((H1, H2), lambda i: (0, 0)),
            pl.BlockSpec((1, H2), lambda i: (0, 0)),
        ],
        out_specs=pl.BlockSpec((1, r, H2), lambda i: (i, 0, 0)),
        out_shape=jax.ShapeDtypeStruct((g, r, H2), jnp.bfloat16),
    )(x, w1.astype(jnp.bfloat16), b1.reshape(1, H1),
      w2.astype(jnp.bfloat16), b2.reshape(1, H2))


def _sites_prenet_body(x_ref, w1_ref, b1_ref, w2_ref, b2_ref,
                       tab_ref, s1_ref):
    ys = []
    for b in range(B):
        x = x_ref[b].astype(jnp.bfloat16)
        h = _relu(_mm(x, w1_ref[...]) + b1_ref[...]).astype(jnp.bfloat16)
        y = _relu(_mm(h, w2_ref[...]) + b2_ref[...]).astype(jnp.bfloat16)
        s1_ref[b] = y
        ys.append(y)
    for b in range(2):
        hi = jax.lax.bitcast_convert_type(ys[b], jnp.uint16).astype(jnp.uint32)
        lo = jax.lax.bitcast_convert_type(ys[b + 2], jnp.uint16).astype(jnp.uint32)
        w = (hi << 16) | lo
        tab_ref[:, b * D:(b + 1) * D] = jax.lax.bitcast_convert_type(w, jnp.int32)


def _run_sites_prenet(sites, wb):
    (w1, b1), (w2, b2) = wb
    return pl.pallas_call(
        _sites_prenet_body,
        in_specs=[
            pl.BlockSpec((B, N, D), lambda: (0, 0, 0)),
            pl.BlockSpec((D, H1), lambda: (0, 0)),
            pl.BlockSpec((1, H1), lambda: (0, 0)),
            pl.BlockSpec((H1, H2), lambda: (0, 0)),
            pl.BlockSpec((1, H2), lambda: (0, 0)),
        ],
        out_specs=[
            pl.BlockSpec((N, F // 2), lambda: (0, 0)),
            pl.BlockSpec((B, N, H2), lambda: (0, 0, 0)),
        ],
        out_shape=[
            jax.ShapeDtypeStruct((N, F // 2), jnp.int32),
            jax.ShapeDtypeStruct((B, N, H2), jnp.bfloat16),
        ],
    )(sites, w1.astype(jnp.bfloat16), b1.reshape(1, H1),
      w2.astype(jnp.bfloat16), b2.reshape(1, H2))


def _sc_gather_body(nrows, table_hbm, idx_hbm, out_hbm, idx_v, rows_v, sem):
    wid = lax.axis_index("s") * NC + lax.axis_index("c")
    rows_per_w = nrows // NW
    base = wid * rows_per_w
    for k in range(rows_per_w // GC):
        start = base + k * GC
        pltpu.sync_copy(idx_hbm.at[pl.ds(start, GC)], idx_v)
        pltpu.async_copy(table_hbm.at[idx_v], rows_v, sem).wait()
        pltpu.sync_copy(rows_v, out_hbm.at[pl.ds(start, GC)])


def _sc_gather(table, idx_cat):
    nrows = idx_cat.shape[0]
    mesh = plsc.VectorSubcoreMesh(core_axis_name="c", subcore_axis_name="s",
                                  num_cores=NC, num_subcores=NS)
    fn = pl.kernel(
        functools.partial(_sc_gather_body, nrows),
        out_type=jax.ShapeDtypeStruct((nrows, F // 2), jnp.int32),
        mesh=mesh,
        scratch_types=[
            pltpu.VMEM((GC,), jnp.int32),
            pltpu.VMEM((GC, F // 2), jnp.int32),
            pltpu.SemaphoreType.DMA,
        ],
    )
    return fn(table, idx_cat)


def _edge_body(bonds_ref, s1_ref, s2_ref, st1_ref,
               wb1_ref, bb1_ref, wb2_ref, bb2_ref,
               wm1_ref, bm1_ref, wm2_ref, bm2_ref, wm3_ref, bm3_ref,
               outb_ref, b2t_ref, esum_ref):
    parts = []
    for b in range(B):
        x = bonds_ref[b]
        xb = x.astype(jnp.bfloat16)
        h = _relu(_mm(xb, wb1_ref[...]) + bb1_ref[...]).astype(jnp.bfloat16)
        bonds1 = _relu(_mm(h, wb2_ref[...]) + bb2_ref[...]).astype(jnp.bfloat16)
        cols = pl.ds((b % 2) * D, D)
        if b < 2:
            mask = jnp.int32(-65536)
            s1 = jax.lax.bitcast_convert_type(s1_ref[:, cols] & mask, jnp.float32)
            s2 = jax.lax.bitcast_convert_type(s2_ref[:, cols] & mask, jnp.float32)
        else:
            s1 = jax.lax.bitcast_convert_type(s1_ref[:, cols] << 16, jnp.float32)
            s2 = jax.lax.bitcast_convert_type(s2_ref[:, cols] << 16, jnp.float32)
        s1 = s1.astype(jnp.bfloat16)
        s2 = s2.astype(jnp.bfloat16)
        sconst = _mm(st1_ref[b:b + 1, :], wm1_ref[3 * H2:4 * H2, :])
        t = (_mm(s1, wm1_ref[0:H2, :]) + _mm(s2, wm1_ref[H2:2 * H2, :])
             + _mm(bonds1, wm1_ref[2 * H2:3 * H2, :]) + sconst + bm1_ref[...])
        t = _relu(t).astype(jnp.bfloat16)
        t = _relu(_mm(t, wm2_ref[...]) + bm2_ref[...]).astype(jnp.bfloat16)
        b2 = _mm(t, wm3_ref[...]) + bm3_ref[...]
        outb_ref[b] = x + b2
        b2t_ref[:, b, :] = b2
        parts.append(jnp.sum(b2, axis=0, keepdims=True))
    b2t_ref[:, B, :] = jnp.ones((BE, D), jnp.float32)
    parts.append(jnp.zeros((8 - B, H2), jnp.float32))
    psum = jnp.concatenate(parts, axis=0)

    @pl.when(pl.program_id(0) == 0)
    def _init():
        esum_ref[...] = psum

    @pl.when(pl.program_id(0) != 0)
    def _acc():
        esum_ref[...] = esum_ref[...] + psum


def _run_edge(bonds, g, states1_pad, params, off, ne):
    (wb1, bb1), (wb2, bb2) = params['bonds_fc']
    (wm1, bm1), (wm2, bm2), (wm3, bm3) = params['bond_mlp']
    nblk = ne // BE
    oblk = off // BE
    return pl.pallas_call(
        _edge_body,
        grid=(nblk,),
        in_specs=[
            pl.BlockSpec((B, BE, D), lambda e: (0, e + oblk, 0)),
            pl.BlockSpec((BE, F // 2), lambda e: (e, 0)),
            pl.BlockSpec((BE, F // 2), lambda e: (e + nblk, 0)),
            pl.BlockSpec((8, D), lambda e: (0, 0)),
            pl.BlockSpec((D, H1), lambda e: (0, 0)),
            pl.BlockSpec((1, H1), lambda e: (0, 0)),
            pl.BlockSpec((H1, H2), lambda e: (0, 0)),
            pl.BlockSpec((1, H2), lambda e: (0, 0)),
            pl.BlockSpec((4 * H2, 256), lambda e: (0, 0)),
            pl.BlockSpec((1, 256), lambda e: (0, 0)),
            pl.BlockSpec((256, 256), lambda e: (0, 0)),
            pl.BlockSpec((1, 256), lambda e: (0, 0)),
            pl.BlockSpec((256, H2), lambda e: (0, 0)),
            pl.BlockSpec((1, H2), lambda e: (0, 0)),
        ],
        out_specs=[
            pl.BlockSpec((B, BE, D), lambda e: (0, e, 0)),
            pl.BlockSpec((BE, B + 1, D), lambda e: (e, 0, 0)),
            pl.BlockSpec((8, H2), lambda e: (0, 0)),
        ],
        out_shape=[
            jax.ShapeDtypeStruct((B, ne, D), jnp.float32),
            jax.ShapeDtypeStruct((ne, B + 1, D), jnp.float32),
            jax.ShapeDtypeStruct((8, H2), jnp.float32),
        ],
    )(bonds, g, g, states1_pad,
      wb1.astype(jnp.bfloat16), bb1.reshape(1, H1),
      wb2.astype(jnp.bfloat16), bb2.reshape(1, H2),
      wm1.astype(jnp.bfloat16), bm1.reshape(1, 256),
      wm2.astype(jnp.bfloat16), bm2.reshape(1, 256),
      wm3.astype(jnp.bfloat16), bm3.reshape(1, H2))


FS = (B + 1) * D


def _sc_scatter_body(ne, b2t_hbm, idx_hbm, zrow_hbm, pool_hbm,
                     rows_v, idx_v, lsem, isem, ssem):
    cid = lax.axis_index("c")
    sid = lax.axis_index("s")
    wid = sid * NC + cid
    zrows = N // NS
    r0 = sid * zrows
    pltpu.sync_copy(zrow_hbm, rows_v.at[0])
    zds = [pltpu.async_copy(rows_v.at[0],
                            pool_hbm.at[cid, pl.ds(r0 + t * 32, 32)], lsem)
           for t in range(zrows // 32)]
    for d in zds:
        d.wait()
    plsc.subcore_barrier()
    pool_c = pool_hbm.at[cid]
    e_per_w = ne // NW
    base = wid * e_per_w
    nch = e_per_w // SC_CHUNK
    nb = 4
    loads = [None] * nb
    scats = [[] for _ in range(nb)]

    def start_load(k):
        bi = k % nb
        st = base + k * SC_CHUNK
        loads[bi] = (
            pltpu.async_copy(b2t_hbm.at[pl.ds(st, SC_CHUNK)], rows_v.at[bi],
                             lsem),
            pltpu.async_copy(idx_hbm.at[pl.ds(st, SC_CHUNK)], idx_v.at[bi],
                             isem),
        )

    start_load(0)
    start_load(1)
    for k in range(nch):
        bi = k % nb
        for d in loads[bi]:
            d.wait()
        for j in range(SC_CHUNK // 16):
            idx_vec = idx_v[bi, pl.ds(j * 16, 16)]
            scats[bi].append(
                pltpu.async_copy(rows_v.at[bi, pl.ds(j * 16, 16)],
                                 pool_c.at[idx_vec], ssem.at[bi], add=True))
        if k + 2 < nch:
            nbi = (k + 2) % nb
            for d in scats[nbi]:
                d.wait()
            scats[nbi] = []
            start_load(k + 2)
    for bl in scats:
        for d in bl:
            d.wait()


def _sc_scatter(b2t, idx1):
    ne = idx1.shape[0]
    mesh = plsc.VectorSubcoreMesh(core_axis_name="c", subcore_axis_name="s",
                                  num_cores=NC, num_subcores=NS)
    fn = pl.kernel(
        functools.partial(_sc_scatter_body, ne),
        out_type=jax.ShapeDtypeStruct((NC, N, FS), jnp.float32),
        mesh=mesh,
        scratch_types=[
            pltpu.VMEM((4, SC_CHUNK, FS), jnp.float32),
            pltpu.VMEM((4, SC_CHUNK), jnp.int32),
            pltpu.SemaphoreType.DMA,
            pltpu.SemaphoreType.DMA,
            pltpu.SemaphoreType.DMA((4,)),
        ],
    )
    zrow = jnp.zeros((32, FS), jnp.float32)
    return fn(b2t, idx1, zrow)


def _site_body(pool_ref, poolb_ref, cnt_ref, cntb_ref, sites1_ref, sites_ref, st1row_ref,
               st1_ref, stpad_ref, esum_ref,
               ws1_ref, bs1_ref, ws2_ref, bs2_ref, ws3_ref, bs3_ref,
               wt1_ref, bt1_ref, wt2_ref, bt2_ref, wt3_ref, bt3_ref,
               osites_ref, ostates_ref, smean_ref):
    b = pl.program_id(0)
    psum = pool_ref[0, :, 0, 0, :] + pool_ref[1, :, 0, 0, :]
    c = cnt_ref[0, :, 0, 0, 0:1] + cnt_ref[1, :, 0, 0, 0:1]
    pool = (psum / jnp.maximum(c, 1.0)).astype(jnp.bfloat16)
    s1b = sites1_ref[0]
    sconst = _mm(st1row_ref[0, 0:1, :], ws1_ref[2 * H2:3 * H2, :])
    t = _relu(_mm(pool, ws1_ref[0:H2, :]) + _mm(s1b, ws1_ref[H2:2 * H2, :])
              + sconst + bs1_ref[...]).astype(jnp.bfloat16)
    t = _relu(_mm(t, ws2_ref[...]) + bs2_ref[...]).astype(jnp.bfloat16)
    s2out = _relu(_mm(t, ws3_ref[...]) + bs3_ref[...])
    osites_ref[0] = sites_ref[0] + s2out

    mean_row = jnp.sum(s2out, axis=0, keepdims=True) / float(N)
    rows = lax.broadcasted_iota(jnp.int32, (8, H2), 0)
    contrib = jnp.where(rows == b, jnp.broadcast_to(mean_row, (8, H2)), 0.0)

    @pl.when(b == 0)
    def _init():
        smean_ref[...] = contrib

    @pl.when(b != 0)
    def _acc():
        smean_ref[...] = smean_ref[...] + contrib

    @pl.when(b == B - 1)
    def _states():
        bmean = (esum_ref[...] / float(E)).astype(jnp.bfloat16)
        v = (_mm(bmean, wt1_ref[0:H2, :])
             + _mm(smean_ref[...].astype(jnp.bfloat16), wt1_ref[H2:2 * H2, :])
             + _mm(st1_ref[...], wt1_ref[2 * H2:3 * H2, :]) + bt1_ref[...])
        v = _relu(v).astype(jnp.bfloat16)
        v = _relu(_mm(v, wt2_ref[...]) + bt2_ref[...]).astype(jnp.bfloat16)
        v = _relu(_mm(v, wt3_ref[...]) + bt3_ref[...])
        ostates_ref[...] = stpad_ref[...] + v


def _run_site(poola, poolb, sites1, sites, states1_pad, states_pad, esum, params):
    (ws1, bs1), (ws2, bs2), (ws3, bs3) = params['site_mlp']
    (wt1, bt1), (wt2, bt2), (wt3, bt3) = params['state_mlp']
    pool5 = poola.reshape(NC, N, B + 1, 1, D)
    pool5b = poolb.reshape(NC, N, B + 1, 1, D)
    st1rows = states1_pad.reshape(8, 1, D)
    return pl.pallas_call(
        _site_body,
        grid=(B,),
        in_specs=[
            pl.BlockSpec((NC, N, 1, 1, D), lambda b: (0, 0, b, 0, 0)),
            pl.BlockSpec((NC, N, 1, 1, D), lambda b: (0, 0, b, 0, 0)),
            pl.BlockSpec((NC, N, 1, 1, D), lambda b: (0, 0, B, 0, 0)),
            pl.BlockSpec((NC, N, 1, 1, D), lambda b: (0, 0, B, 0, 0)),
            pl.BlockSpec((1, N, D), lambda b: (b, 0, 0)),
            pl.BlockSpec((1, N, D), lambda b: (b, 0, 0)),
            pl.BlockSpec((1, 1, D), lambda b: (b, 0, 0)),
            pl.BlockSpec((8, D), lambda b: (0, 0)),
            pl.BlockSpec((8, D), lambda b: (0, 0)),
            pl.BlockSpec((8, H2), lambda b: (0, 0)),
            pl.BlockSpec((3 * H2, 256), lambda b: (0, 0)),
            pl.BlockSpec((1, 256), lambda b: (0, 0)),
            pl.BlockSpec((256, 256), lambda b: (0, 0)),
            pl.BlockSpec((1, 256), lambda b: (0, 0)),
            pl.BlockSpec((256, H2), lambda b: (0, 0)),
            pl.BlockSpec((1, H2), lambda b: (0, 0)),
            pl.BlockSpec((3 * H2, 256), lambda b: (0, 0)),
            pl.BlockSpec((1, 256), lambda b: (0, 0)),
            pl.BlockSpec((256, 256), lambda b: (0, 0)),
            pl.BlockSpec((1, 256), lambda b: (0, 0)),
            pl.BlockSpec((256, H2), lambda b: (0, 0)),
            pl.BlockSpec((1, H2), lambda b: (0, 0)),
        ],
        out_specs=[
            pl.BlockSpec((1, N, D), lambda b: (b, 0, 0)),
            pl.BlockSpec((8, D), lambda b: (0, 0)),
        ],
        out_shape=[
            jax.ShapeDtypeStruct((B, N, D), jnp.float32),
            jax.ShapeDtypeStruct((8, D), jnp.float32),
        ],
        scratch_shapes=[pltpu.VMEM((8, H2), jnp.float32)],
    )(pool5, pool5b, pool5, pool5b, sites1, sites, st1rows, states1_pad, states_pad, esum,
      ws1.astype(jnp.bfloat16), bs1.reshape(1, 256),
      ws2.astype(jnp.bfloat16), bs2.reshape(1, 256),
      ws3.astype(jnp.bfloat16), bs3.reshape(1, H2),
      wt1.astype(jnp.bfloat16), bt1.reshape(1, 256),
      wt2.astype(jnp.bfloat16), bt2.reshape(1, 256),
      wt3.astype(jnp.bfloat16), bt3.reshape(1, H2))


def kernel(sites, bonds, states, indices1, indices2, params):
    idx1 = indices1.astype(jnp.int32)
    idx2 = indices2.astype(jnp.int32)
    states_pad = jnp.pad(states, ((0, 8 - B), (0, 0)))

    states1_pad = _run_prenet(states_pad.reshape(8, 1, D),
                              params['states_fc']).reshape(8, H2)
    table, sites1 = _run_sites_prenet(sites, params['sites_fc'])

    g = _sc_gather(table, jnp.concatenate([idx1, idx2]))
    bonds_out, b2t, esum = _run_edge(bonds, g, states1_pad, params, 0, E)
    pool = _sc_scatter(b2t.reshape(E, FS), idx1)
    sites_out, states_out_pad = _run_site(pool, pool, sites1, sites,
                                          states1_pad, states_pad,
                                          esum, params)
    return sites_out, bonds_out, states_out_pad[:B]


# speedup vs baseline: 1.6794x; 1.0825x over previous
import functools

import jax
import jax.numpy as jnp
from jax import lax
from jax.experimental import pallas as pl
from jax.experimental.pallas import tpu as pltpu
from jax.experimental.pallas import tpu_sc as plsc

B, N, E, D = 4, 2048, 32768, 128
H1, H2 = 256, 128
NC, NS = 2, 16
NW = NC * NS
GC = 128
SC_CHUNK = 32
BE = 512
F = B * D


def _relu(x):
    return jnp.maximum(x, 0.0)


def _mm(x, w):
    return jax.lax.dot_general(x, w, (((x.ndim - 1,), (0,)), ((), ())),
                               preferred_element_type=jnp.float32)


def _prenet_body(x_ref, w1_ref, b1_ref, w2_ref, b2_ref, o_ref):
    x = x_ref[0].astype(jnp.bfloat16)
    h = _relu(_mm(x, w1_ref[...]) + b1_ref[...]).astype(jnp.bfloat16)
    o_ref[0] = _relu(_mm(h, w2_ref[...]) + b2_ref[...]).astype(jnp.bfloat16)


def _run_prenet(x, wb):
    (w1, b1), (w2, b2) = wb
    g, r, d = x.shape
    return pl.pallas_call(
        _prenet_body,
        grid=(g,),
        in_specs=[
            pl.BlockSpec((1, r, d), lambda i: (i, 0, 0)),
            pl.BlockSpec((d, H1), lambda i: (0, 0)),
            pl.BlockSpec((1, H1), lambda i: (0, 0)),
            pl.BlockSpec((H1, H2), lambda i: (0, 0)),
            pl.BlockSpec((1, H2), lambda i: (0, 0)),
        ],
        out_specs=pl.BlockSpec((1, r, H2), lambda i: (i, 0, 0)),
        out_shape=jax.ShapeDtypeStruct((g, r, H2), jnp.bfloat16),
    )(x, w1.astype(jnp.bfloat16), b1.reshape(1, H1),
      w2.astype(jnp.bfloat16), b2.reshape(1, H2))


def _sites_prenet_body(x_ref, w1_ref, b1_ref, w2_ref, b2_ref,
                       tab_ref, s1_ref):
    ys = []
    for b in range(B):
        x = x_ref[b].astype(jnp.bfloat16)
        h = _relu(_mm(x, w1_ref[...]) + b1_ref[...]).astype(jnp.bfloat16)
        y = _relu(_mm(h, w2_ref[...]) + b2_ref[...]).astype(jnp.bfloat16)
        s1_ref[b] = y
        ys.append(y)
    for b in range(2):
        hi = jax.lax.bitcast_convert_type(ys[b], jnp.uint16).astype(jnp.uint32)
        lo = jax.lax.bitcast_convert_type(ys[b + 2], jnp.uint16).astype(jnp.uint32)
        w = (hi << 16) | lo
        tab_ref[:, b * D:(b + 1) * D] = jax.lax.bitcast_convert_type(w, jnp.int32)


def _run_sites_prenet(sites, wb):
    (w1, b1), (w2, b2) = wb
    return pl.pallas_call(
        _sites_prenet_body,
        in_specs=[
            pl.BlockSpec((B, N, D), lambda: (0, 0, 0)),
            pl.BlockSpec((D, H1), lambda: (0, 0)),
            pl.BlockSpec((1, H1), lambda: (0, 0)),
            pl.BlockSpec((H1, H2), lambda: (0, 0)),
            pl.BlockSpec((1, H2), lambda: (0, 0)),
        ],
        out_specs=[
            pl.BlockSpec((N, F // 2), lambda: (0, 0)),
            pl.BlockSpec((B, N, H2), lambda: (0, 0, 0)),
        ],
        out_shape=[
            jax.ShapeDtypeStruct((N, F // 2), jnp.int32),
            jax.ShapeDtypeStruct((B, N, H2), jnp.bfloat16),
        ],
    )(sites, w1.astype(jnp.bfloat16), b1.reshape(1, H1),
      w2.astype(jnp.bfloat16), b2.reshape(1, H2))


def _sc_gather_body(nrows, table_hbm, idx_hbm, out_hbm, idx_v, rows_v, sem):
    wid = lax.axis_index("s") * NC + lax.axis_index("c")
    rows_per_w = nrows // NW
    base = wid * rows_per_w
    for k in range(rows_per_w // GC):
        start = base + k * GC
        pltpu.sync_copy(idx_hbm.at[pl.ds(start, GC)], idx_v)
        pltpu.async_copy(table_hbm.at[idx_v], rows_v, sem).wait()
        pltpu.sync_copy(rows_v, out_hbm.at[pl.ds(start, GC)])


def _sc_gather(table, idx_cat):
    nrows = idx_cat.shape[0]
    mesh = plsc.VectorSubcoreMesh(core_axis_name="c", subcore_axis_name="s",
                                  num_cores=NC, num_subcores=NS)
    fn = pl.kernel(
        functools.partial(_sc_gather_body, nrows),
        out_type=jax.ShapeDtypeStruct((nrows, F // 2), jnp.int32),
        mesh=mesh,
        scratch_types=[
            pltpu.VMEM((GC,), jnp.int32),
            pltpu.VMEM((GC, F // 2), jnp.int32),
            pltpu.SemaphoreType.DMA,
        ],
    )
    return fn(table, idx_cat)


def _edge_body(bonds_ref, s1_ref, s2_ref, st1_ref,
               wb1_ref, bb1_ref, wb2_ref, bb2_ref,
               wm1_ref, bm1_ref, wm2_ref, bm2_ref, wm3_ref, bm3_ref,
               outb_ref, b2t_ref, esum_ref):
    x_all = bonds_ref[...].reshape(B * BE, D)
    xb = x_all.astype(jnp.bfloat16)
    h = _relu(_mm(xb, wb1_ref[...]) + bb1_ref[...]).astype(jnp.bfloat16)
    bonds1 = _relu(_mm(h, wb2_ref[...]) + bb2_ref[...]).astype(jnp.bfloat16)

    mask = jnp.int32(-65536)

    def unpack(u_ref):
        u = u_ref[...]
        hi = jax.lax.bitcast_convert_type(u & mask, jnp.float32)
        lo = jax.lax.bitcast_convert_type(u << 16, jnp.float32)
        return jnp.concatenate([hi[:, 0:D], hi[:, D:2 * D],
                                lo[:, 0:D], lo[:, D:2 * D]],
                               axis=0).astype(jnp.bfloat16)

    s1_all = unpack(s1_ref)
    s2_all = unpack(s2_ref)
    t = (_mm(s1_all, wm1_ref[0:H2, :]) + _mm(s2_all, wm1_ref[H2:2 * H2, :])
         + _mm(bonds1, wm1_ref[2 * H2:3 * H2, :]))
    sconst = _mm(st1_ref[0:B, :], wm1_ref[3 * H2:4 * H2, :])
    cadd = jnp.concatenate(
        [jnp.broadcast_to(sconst[b:b + 1, :], (BE, 256)) for b in range(B)],
        axis=0)
    t = _relu(t + cadd + bm1_ref[...]).astype(jnp.bfloat16)
    t = _relu(_mm(t, wm2_ref[...]) + bm2_ref[...]).astype(jnp.bfloat16)
    b2 = _mm(t, wm3_ref[...]) + bm3_ref[...]
    outb_ref[...] = (x_all + b2).reshape(B, BE, D)
    parts = []
    for b in range(B):
        blk = b2[b * BE:(b + 1) * BE, :]
        b2t_ref[:, b, :] = blk
        parts.append(jnp.sum(blk, axis=0, keepdims=True))
    b2t_ref[:, B, :] = jnp.ones((BE, D), jnp.float32)
    parts.append(jnp.zeros((8 - B, H2), jnp.float32))
    psum = jnp.concatenate(parts, axis=0)

    @pl.when(pl.program_id(0) == 0)
    def _init():
        esum_ref[...] = psum

    @pl.when(pl.program_id(0) != 0)
    def _acc():
        esum_ref[...] = esum_ref[...] + psum


def _run_edge(bonds, g, states1_pad, params, off, ne):
    (wb1, bb1), (wb2, bb2) = params['bonds_fc']
    (wm1, bm1), (wm2, bm2), (wm3, bm3) = params['bond_mlp']
    nblk = ne // BE
    oblk = off // BE
    return pl.pallas_call(
        _edge_body,
        grid=(nblk,),
        in_specs=[
            pl.BlockSpec((B, BE, D), lambda e: (0, e + oblk, 0)),
            pl.BlockSpec((BE, F // 2), lambda e: (e, 0)),
            pl.BlockSpec((BE, F // 2), lambda e: (e + nblk, 0)),
            pl.BlockSpec((8, D), lambda e: (0, 0)),
            pl.BlockSpec((D, H1), lambda e: (0, 0)),
            pl.BlockSpec((1, H1), lambda e: (0, 0)),
            pl.BlockSpec((H1, H2), lambda e: (0, 0)),
            pl.BlockSpec((1, H2), lambda e: (0, 0)),
            pl.BlockSpec((4 * H2, 256), lambda e: (0, 0)),
            pl.BlockSpec((1, 256), lambda e: (0, 0)),
            pl.BlockSpec((256, 256), lambda e: (0, 0)),
            pl.BlockSpec((1, 256), lambda e: (0, 0)),
            pl.BlockSpec((256, H2), lambda e: (0, 0)),
            pl.BlockSpec((1, H2), lambda e: (0, 0)),
        ],
        out_specs=[
            pl.BlockSpec((B, BE, D), lambda e: (0, e, 0)),
            pl.BlockSpec((BE, B + 1, D), lambda e: (e, 0, 0)),
            pl.BlockSpec((8, H2), lambda e: (0, 0)),
        ],
        out_shape=[
            jax.ShapeDtypeStruct((B, ne, D), jnp.float32),
            jax.ShapeDtypeStruct((ne, B + 1, D), jnp.float32),
            jax.ShapeDtypeStruct((8, H2), jnp.float32),
        ],
    )(bonds, g, g, states1_pad,
      wb1.astype(jnp.bfloat16), bb1.reshape(1, H1),
      wb2.astype(jnp.bfloat16), bb2.reshape(1, H2),
      wm1.astype(jnp.bfloat16), bm1.reshape(1, 256),
      wm2.astype(jnp.bfloat16), bm2.reshape(1, 256),
      wm3.astype(jnp.bfloat16), bm3.reshape(1, H2))


FS = (B + 1) * D


def _sc_scatter_body(ne, b2t_hbm, idx_hbm, zrow_hbm, pool_hbm,
                     rows_v, idx_v, lsem, isem, ssem):
    cid = lax.axis_index("c")
    sid = lax.axis_index("s")
    wid = sid * NC + cid
    zrows = N // NS
    r0 = sid * zrows
    pltpu.sync_copy(zrow_hbm, rows_v.at[0])
    zds = [pltpu.async_copy(rows_v.at[0],
                            pool_hbm.at[cid, pl.ds(r0 + t * 32, 32)], lsem)
           for t in range(zrows // 32)]
    for d in zds:
        d.wait()
    plsc.subcore_barrier()
    pool_c = pool_hbm.at[cid]
    e_per_w = ne // NW
    base = wid * e_per_w
    nch = e_per_w // SC_CHUNK
    nb = 4
    loads = [None] * nb
    scats = [[] for _ in range(nb)]

    def start_load(k):
        bi = k % nb
        st = base + k * SC_CHUNK
        loads[bi] = (
            pltpu.async_copy(b2t_hbm.at[pl.ds(st, SC_CHUNK)], rows_v.at[bi],
                             lsem),
            pltpu.async_copy(idx_hbm.at[pl.ds(st, SC_CHUNK)], idx_v.at[bi],
                             isem),
        )

    start_load(0)
    start_load(1)
    for k in range(nch):
        bi = k % nb
        for d in loads[bi]:
            d.wait()
        for j in range(SC_CHUNK // 16):
            idx_vec = idx_v[bi, pl.ds(j * 16, 16)]
            scats[bi].append(
                pltpu.async_copy(rows_v.at[bi, pl.ds(j * 16, 16)],
                                 pool_c.at[idx_vec], ssem.at[bi], add=True))
        if k + 2 < nch:
            nbi = (k + 2) % nb
            for d in scats[nbi]:
                d.wait()
            scats[nbi] = []
            start_load(k + 2)
    for bl in scats:
        for d in bl:
            d.wait()


def _sc_scatter(b2t, idx1):
    ne = idx1.shape[0]
    mesh = plsc.VectorSubcoreMesh(core_axis_name="c", subcore_axis_name="s",
                                  num_cores=NC, num_subcores=NS)
    fn = pl.kernel(
        functools.partial(_sc_scatter_body, ne),
        out_type=jax.ShapeDtypeStruct((NC, N, FS), jnp.float32),
        mesh=mesh,
        scratch_types=[
            pltpu.VMEM((4, SC_CHUNK, FS), jnp.float32),
            pltpu.VMEM((4, SC_CHUNK), jnp.int32),
            pltpu.SemaphoreType.DMA,
            pltpu.SemaphoreType.DMA,
            pltpu.SemaphoreType.DMA((4,)),
        ],
    )
    zrow = jnp.zeros((32, FS), jnp.float32)
    return fn(b2t, idx1, zrow)


def _site_body(pool_ref, poolb_ref, cnt_ref, cntb_ref, sites1_ref, sites_ref, st1row_ref,
               st1_ref, stpad_ref, esum_ref,
               ws1_ref, bs1_ref, ws2_ref, bs2_ref, ws3_ref, bs3_ref,
               wt1_ref, bt1_ref, wt2_ref, bt2_ref, wt3_ref, bt3_ref,
               osites_ref, ostates_ref, smean_ref):
    b = pl.program_id(0)
    psum = pool_ref[0, :, 0, 0, :] + pool_ref[1, :, 0, 0, :]
    c = cnt_ref[0, :, 0, 0, 0:1] + cnt_ref[1, :, 0, 0, 0:1]
    pool = (psum / jnp.maximum(c, 1.0)).astype(jnp.bfloat16)
    s1b = sites1_ref[0]
    sconst = _mm(st1row_ref[0, 0:1, :], ws1_ref[2 * H2:3 * H2, :])
    t = _relu(_mm(pool, ws1_ref[0:H2, :]) + _mm(s1b, ws1_ref[H2:2 * H2, :])
              + sconst + bs1_ref[...]).astype(jnp.bfloat16)
    t = _relu(_mm(t, ws2_ref[...]) + bs2_ref[...]).astype(jnp.bfloat16)
    s2out = _relu(_mm(t, ws3_ref[...]) + bs3_ref[...])
    osites_ref[0] = sites_ref[0] + s2out

    mean_row = jnp.sum(s2out, axis=0, keepdims=True) / float(N)
    rows = lax.broadcasted_iota(jnp.int32, (8, H2), 0)
    contrib = jnp.where(rows == b, jnp.broadcast_to(mean_row, (8, H2)), 0.0)

    @pl.when(b == 0)
    def _init():
        smean_ref[...] = contrib

    @pl.when(b != 0)
    def _acc():
        smean_ref[...] = smean_ref[...] + contrib

    @pl.when(b == B - 1)
    def _states():
        bmean = (esum_ref[...] / float(E)).astype(jnp.bfloat16)
        v = (_mm(bmean, wt1_ref[0:H2, :])
             + _mm(smean_ref[...].astype(jnp.bfloat16), wt1_ref[H2:2 * H2, :])
             + _mm(st1_ref[...], wt1_ref[2 * H2:3 * H2, :]) + bt1_ref[...])
        v = _relu(v).astype(jnp.bfloat16)
        v = _relu(_mm(v, wt2_ref[...]) + bt2_ref[...]).astype(jnp.bfloat16)
        v = _relu(_mm(v, wt3_ref[...]) + bt3_ref[...])
        ostates_ref[...] = stpad_ref[...] + v


def _run_site(poola, poolb, sites1, sites, states1_pad, states_pad, esum, params):
    (ws1, bs1), (ws2, bs2), (ws3, bs3) = params['site_mlp']
    (wt1, bt1), (wt2, bt2), (wt3, bt3) = params['state_mlp']
    pool5 = poola.reshape(NC, N, B + 1, 1, D)
    pool5b = poolb.reshape(NC, N, B + 1, 1, D)
    st1rows = states1_pad.reshape(8, 1, D)
    return pl.pallas_call(
        _site_body,
        grid=(B,),
        in_specs=[
            pl.BlockSpec((NC, N, 1, 1, D), lambda b: (0, 0, b, 0, 0)),
            pl.BlockSpec((NC, N, 1, 1, D), lambda b: (0, 0, b, 0, 0)),
            pl.BlockSpec((NC, N, 1, 1, D), lambda b: (0, 0, B, 0, 0)),
            pl.BlockSpec((NC, N, 1, 1, D), lambda b: (0, 0, B, 0, 0)),
            pl.BlockSpec((1, N, D), lambda b: (b, 0, 0)),
            pl.BlockSpec((1, N, D), lambda b: (b, 0, 0)),
            pl.BlockSpec((1, 1, D), lambda b: (b, 0, 0)),
            pl.BlockSpec((8, D), lambda b: (0, 0)),
            pl.BlockSpec((8, D), lambda b: (0, 0)),
            pl.BlockSpec((8, H2), lambda b: (0, 0)),
            pl.BlockSpec((3 * H2, 256), lambda b: (0, 0)),
            pl.BlockSpec((1, 256), lambda b: (0, 0)),
            pl.BlockSpec((256, 256), lambda b: (0, 0)),
            pl.BlockSpec((1, 256), lambda b: (0, 0)),
            pl.BlockSpec((256, H2), lambda b: (0, 0)),
            pl.BlockSpec((1, H2), lambda b: (0, 0)),
            pl.BlockSpec((3 * H2, 256), lambda b: (0, 0)),
            pl.BlockSpec((1, 256), lambda b: (0, 0)),
            pl.BlockSpec((256, 256), lambda b: (0, 0)),
            pl.BlockSpec((1, 256), lambda b: (0, 0)),
            pl.BlockSpec((256, H2), lambda b: (0, 0)),
            pl.BlockSpec((1, H2), lambda b: (0, 0)),
        ],
        out_specs=[
            pl.BlockSpec((1, N, D), lambda b: (b, 0, 0)),
            pl.BlockSpec((8, D), lambda b: (0, 0)),
        ],
        out_shape=[
            jax.ShapeDtypeStruct((B, N, D), jnp.float32),
            jax.ShapeDtypeStruct((8, D), jnp.float32),
        ],
        scratch_shapes=[pltpu.VMEM((8, H2), jnp.float32)],
    )(pool5, pool5b, pool5, pool5b, sites1, sites, st1rows, states1_pad, states_pad, esum,
      ws1.astype(jnp.bfloat16), bs1.reshape(1, 256),
      ws2.astype(jnp.bfloat16), bs2.reshape(1, 256),
      ws3.astype(jnp.bfloat16), bs3.reshape(1, H2),
      wt1.astype(jnp.bfloat16), bt1.reshape(1, 256),
      wt2.astype(jnp.bfloat16), bt2.reshape(1, 256),
      wt3.astype(jnp.bfloat16), bt3.reshape(1, H2))


def kernel(sites, bonds, states, indices1, indices2, params):
    idx1 = indices1.astype(jnp.int32)
    idx2 = indices2.astype(jnp.int32)
    states_pad = jnp.pad(states, ((0, 8 - B), (0, 0)))

    states1_pad = _run_prenet(states_pad.reshape(8, 1, D),
                              params['states_fc']).reshape(8, H2)
    table, sites1 = _run_sites_prenet(sites, params['sites_fc'])

    g = _sc_gather(table, jnp.concatenate([idx1, idx2]))
    bonds_out, b2t, esum = _run_edge(bonds, g, states1_pad, params, 0, E)
    pool = _sc_scatter(b2t.reshape(E, FS), idx1)
    sites_out, states_out_pad = _run_site(pool, pool, sites1, sites,
                                          states1_pad, states_pad,
                                          esum, params)
    return sites_out, bonds_out, states_out_pad[:B]


# speedup vs baseline: 1.7251x; 1.0272x over previous
import functools

import jax
import jax.numpy as jnp
from jax import lax
from jax.experimental import pallas as pl
from jax.experimental.pallas import tpu as pltpu
from jax.experimental.pallas import tpu_sc as plsc

B, N, E, D = 4, 2048, 32768, 128
H1, H2 = 256, 128
NC, NS = 2, 16
NW = NC * NS
GC = 128
SC_CHUNK = 32
BE = 1024
F = B * D


def _relu(x):
    return jnp.maximum(x, 0.0)


def _mm(x, w):
    return jax.lax.dot_general(x, w, (((x.ndim - 1,), (0,)), ((), ())),
                               preferred_element_type=jnp.float32)


def _prenet_body(x_ref, w1_ref, b1_ref, w2_ref, b2_ref, o_ref):
    x = x_ref[0].astype(jnp.bfloat16)
    h = _relu(_mm(x, w1_ref[...]) + b1_ref[...]).astype(jnp.bfloat16)
    o_ref[0] = _relu(_mm(h, w2_ref[...]) + b2_ref[...]).astype(jnp.bfloat16)


def _run_prenet(x, wb):
    (w1, b1), (w2, b2) = wb
    g, r, d = x.shape
    return pl.pallas_call(
        _prenet_body,
        grid=(g,),
        in_specs=[
            pl.BlockSpec((1, r, d), lambda i: (i, 0, 0)),
            pl.BlockSpec((d, H1), lambda i: (0, 0)),
            pl.BlockSpec((1, H1), lambda i: (0, 0)),
            pl.BlockSpec((H1, H2), lambda i: (0, 0)),
            pl.BlockSpec((1, H2), lambda i: (0, 0)),
        ],
        out_specs=pl.BlockSpec((1, r, H2), lambda i: (i, 0, 0)),
        out_shape=jax.ShapeDtypeStruct((g, r, H2), jnp.bfloat16),
    )(x, w1.astype(jnp.bfloat16), b1.reshape(1, H1),
      w2.astype(jnp.bfloat16), b2.reshape(1, H2))


def _sites_prenet_body(x_ref, w1_ref, b1_ref, w2_ref, b2_ref,
                       tab_ref, s1_ref):
    ys = []
    for b in range(B):
        x = x_ref[b].astype(jnp.bfloat16)
        h = _relu(_mm(x, w1_ref[...]) + b1_ref[...]).astype(jnp.bfloat16)
        y = _relu(_mm(h, w2_ref[...]) + b2_ref[...]).astype(jnp.bfloat16)
        s1_ref[b] = y
        ys.append(y)
    for b in range(2):
        hi = jax.lax.bitcast_convert_type(ys[b], jnp.uint16).astype(jnp.uint32)
        lo = jax.lax.bitcast_convert_type(ys[b + 2], jnp.uint16).astype(jnp.uint32)
        w = (hi << 16) | lo
        tab_ref[:, b * D:(b + 1) * D] = jax.lax.bitcast_convert_type(w, jnp.int32)


def _run_sites_prenet(sites, wb):
    (w1, b1), (w2, b2) = wb
    return pl.pallas_call(
        _sites_prenet_body,
        in_specs=[
            pl.BlockSpec((B, N, D), lambda: (0, 0, 0)),
            pl.BlockSpec((D, H1), lambda: (0, 0)),
            pl.BlockSpec((1, H1), lambda: (0, 0)),
            pl.BlockSpec((H1, H2), lambda: (0, 0)),
            pl.BlockSpec((1, H2), lambda: (0, 0)),
        ],
        out_specs=[
            pl.BlockSpec((N, F // 2), lambda: (0, 0)),
            pl.BlockSpec((B, N, H2), lambda: (0, 0, 0)),
        ],
        out_shape=[
            jax.ShapeDtypeStruct((N, F // 2), jnp.int32),
            jax.ShapeDtypeStruct((B, N, H2), jnp.bfloat16),
        ],
    )(sites, w1.astype(jnp.bfloat16), b1.reshape(1, H1),
      w2.astype(jnp.bfloat16), b2.reshape(1, H2))


def _sc_gather_body(nrows, table_hbm, idx_hbm, out_hbm, idx_v, rows_v, sem):
    wid = lax.axis_index("s") * NC + lax.axis_index("c")
    rows_per_w = nrows // NW
    base = wid * rows_per_w
    for k in range(rows_per_w // GC):
        start = base + k * GC
        pltpu.sync_copy(idx_hbm.at[pl.ds(start, GC)], idx_v)
        pltpu.async_copy(table_hbm.at[idx_v], rows_v, sem).wait()
        pltpu.sync_copy(rows_v, out_hbm.at[pl.ds(start, GC)])


def _sc_gather(table, idx_cat):
    nrows = idx_cat.shape[0]
    mesh = plsc.VectorSubcoreMesh(core_axis_name="c", subcore_axis_name="s",
                                  num_cores=NC, num_subcores=NS)
    fn = pl.kernel(
        functools.partial(_sc_gather_body, nrows),
        out_type=jax.ShapeDtypeStruct((nrows, F // 2), jnp.int32),
        mesh=mesh,
        scratch_types=[
            pltpu.VMEM((GC,), jnp.int32),
            pltpu.VMEM((GC, F // 2), jnp.int32),
            pltpu.SemaphoreType.DMA,
        ],
    )
    return fn(table, idx_cat)


def _edge_body(bonds_ref, s1_ref, s2_ref, st1_ref,
               wb1_ref, bb1_ref, wb2_ref, bb2_ref,
               wm1_ref, bm1_ref, wm2_ref, bm2_ref, wm3_ref, bm3_ref,
               outb_ref, b2t_ref, esum_ref):
    x_all = bonds_ref[...].reshape(B * BE, D)
    xb = x_all.astype(jnp.bfloat16)
    h = _relu(_mm(xb, wb1_ref[...]) + bb1_ref[...]).astype(jnp.bfloat16)
    bonds1 = _relu(_mm(h, wb2_ref[...]) + bb2_ref[...]).astype(jnp.bfloat16)

    mask = jnp.int32(-65536)

    def unpack(u_ref):
        u = u_ref[...]
        hi = jax.lax.bitcast_convert_type(u & mask, jnp.float32)
        lo = jax.lax.bitcast_convert_type(u << 16, jnp.float32)
        return jnp.concatenate([hi[:, 0:D], hi[:, D:2 * D],
                                lo[:, 0:D], lo[:, D:2 * D]],
                               axis=0).astype(jnp.bfloat16)

    s1_all = unpack(s1_ref)
    s2_all = unpack(s2_ref)
    t = (_mm(s1_all, wm1_ref[0:H2, :]) + _mm(s2_all, wm1_ref[H2:2 * H2, :])
         + _mm(bonds1, wm1_ref[2 * H2:3 * H2, :]))
    sconst = _mm(st1_ref[0:B, :], wm1_ref[3 * H2:4 * H2, :])
    cadd = jnp.concatenate(
        [jnp.broadcast_to(sconst[b:b + 1, :], (BE, 256)) for b in range(B)],
        axis=0)
    t = _relu(t + cadd + bm1_ref[...]).astype(jnp.bfloat16)
    t = _relu(_mm(t, wm2_ref[...]) + bm2_ref[...]).astype(jnp.bfloat16)
    b2 = _mm(t, wm3_ref[...]) + bm3_ref[...]
    outb_ref[...] = (x_all + b2).reshape(B, BE, D)
    parts = []
    for b in range(B):
        blk = b2[b * BE:(b + 1) * BE, :]
        b2t_ref[:, b, :] = blk
        parts.append(jnp.sum(blk, axis=0, keepdims=True))
    b2t_ref[:, B, :] = jnp.ones((BE, D), jnp.float32)
    parts.append(jnp.zeros((8 - B, H2), jnp.float32))
    psum = jnp.concatenate(parts, axis=0)

    @pl.when(pl.program_id(0) == 0)
    def _init():
        esum_ref[...] = psum

    @pl.when(pl.program_id(0) != 0)
    def _acc():
        esum_ref[...] = esum_ref[...] + psum


def _run_edge(bonds, g, states1_pad, params, off, ne):
    (wb1, bb1), (wb2, bb2) = params['bonds_fc']
    (wm1, bm1), (wm2, bm2), (wm3, bm3) = params['bond_mlp']
    nblk = ne // BE
    oblk = off // BE
    return pl.pallas_call(
        _edge_body,
        grid=(nblk,),
        in_specs=[
            pl.BlockSpec((B, BE, D), lambda e: (0, e + oblk, 0)),
            pl.BlockSpec((BE, F // 2), lambda e: (e, 0)),
            pl.BlockSpec((BE, F // 2), lambda e: (e + nblk, 0)),
            pl.BlockSpec((8, D), lambda e: (0, 0)),
            pl.BlockSpec((D, H1), lambda e: (0, 0)),
            pl.BlockSpec((1, H1), lambda e: (0, 0)),
            pl.BlockSpec((H1, H2), lambda e: (0, 0)),
            pl.BlockSpec((1, H2), lambda e: (0, 0)),
            pl.BlockSpec((4 * H2, 256), lambda e: (0, 0)),
            pl.BlockSpec((1, 256), lambda e: (0, 0)),
            pl.BlockSpec((256, 256), lambda e: (0, 0)),
            pl.BlockSpec((1, 256), lambda e: (0, 0)),
            pl.BlockSpec((256, H2), lambda e: (0, 0)),
            pl.BlockSpec((1, H2), lambda e: (0, 0)),
        ],
        out_specs=[
            pl.BlockSpec((B, BE, D), lambda e: (0, e, 0)),
            pl.BlockSpec((BE, B + 1, D), lambda e: (e, 0, 0)),
            pl.BlockSpec((8, H2), lambda e: (0, 0)),
        ],
        out_shape=[
            jax.ShapeDtypeStruct((B, ne, D), jnp.float32),
            jax.ShapeDtypeStruct((ne, B + 1, D), jnp.float32),
            jax.ShapeDtypeStruct((8, H2), jnp.float32),
        ],
    )(bonds, g, g, states1_pad,
      wb1.astype(jnp.bfloat16), bb1.reshape(1, H1),
      wb2.astype(jnp.bfloat16), bb2.reshape(1, H2),
      wm1.astype(jnp.bfloat16), bm1.reshape(1, 256),
      wm2.astype(jnp.bfloat16), bm2.reshape(1, 256),
      wm3.astype(jnp.bfloat16), bm3.reshape(1, H2))


FS = (B + 1) * D


def _sc_scatter_body(ne, b2t_hbm, idx_hbm, zrow_hbm, pool_hbm,
                     rows_v, idx_v, lsem, isem, ssem):
    cid = lax.axis_index("c")
    sid = lax.axis_index("s")
    wid = sid * NC + cid
    zrows = N // NS
    r0 = sid * zrows
    pltpu.sync_copy(zrow_hbm, rows_v.at[0])
    zds = [pltpu.async_copy(rows_v.at[0],
                            pool_hbm.at[cid, pl.ds(r0 + t * 32, 32)], lsem)
           for t in range(zrows // 32)]
    for d in zds:
        d.wait()
    plsc.subcore_barrier()
    pool_c = pool_hbm.at[cid]
    e_per_w = ne // NW
    base = wid * e_per_w
    nch = e_per_w // SC_CHUNK
    nb = 4
    loads = [None] * nb
    scats = [[] for _ in range(nb)]

    def start_load(k):
        bi = k % nb
        st = base + k * SC_CHUNK
        loads[bi] = (
            pltpu.async_copy(b2t_hbm.at[pl.ds(st, SC_CHUNK)], rows_v.at[bi],
                             lsem),
            pltpu.async_copy(idx_hbm.at[pl.ds(st, SC_CHUNK)], idx_v.at[bi],
                             isem),
        )

    start_load(0)
    start_load(1)
    for k in range(nch):
        bi = k % nb
        for d in loads[bi]:
            d.wait()
        for j in range(SC_CHUNK // 16):
            idx_vec = idx_v[bi, pl.ds(j * 16, 16)]
            scats[bi].append(
                pltpu.async_copy(rows_v.at[bi, pl.ds(j * 16, 16)],
                                 pool_c.at[idx_vec], ssem.at[bi], add=True))
        if k + 2 < nch:
            nbi = (k + 2) % nb
            for d in scats[nbi]:
                d.wait()
            scats[nbi] = []
            start_load(k + 2)
    for bl in scats:
        for d in bl:
            d.wait()


def _sc_scatter(b2t, idx1):
    ne = idx1.shape[0]
    mesh = plsc.VectorSubcoreMesh(core_axis_name="c", subcore_axis_name="s",
                                  num_cores=NC, num_subcores=NS)
    fn = pl.kernel(
        functools.partial(_sc_scatter_body, ne),
        out_type=jax.ShapeDtypeStruct((NC, N, FS), jnp.float32),
        mesh=mesh,
        scratch_types=[
            pltpu.VMEM((4, SC_CHUNK, FS), jnp.float32),
            pltpu.VMEM((4, SC_CHUNK), jnp.int32),
            pltpu.SemaphoreType.DMA,
            pltpu.SemaphoreType.DMA,
            pltpu.SemaphoreType.DMA((4,)),
        ],
    )
    zrow = jnp.zeros((32, FS), jnp.float32)
    return fn(b2t, idx1, zrow)


def _site_body(pool_ref, poolb_ref, cnt_ref, cntb_ref, sites1_ref, sites_ref, st1row_ref,
               st1_ref, stpad_ref, esum_ref,
               ws1_ref, bs1_ref, ws2_ref, bs2_ref, ws3_ref, bs3_ref,
               wt1_ref, bt1_ref, wt2_ref, bt2_ref, wt3_ref, bt3_ref,
               osites_ref, ostates_ref, smean_ref):
    b = pl.program_id(0)
    psum = pool_ref[0, :, 0, 0, :] + pool_ref[1, :, 0, 0, :]
    c = cnt_ref[0, :, 0, 0, 0:1] + cnt_ref[1, :, 0, 0, 0:1]
    pool = (psum / jnp.maximum(c, 1.0)).astype(jnp.bfloat16)
    s1b = sites1_ref[0]
    sconst = _mm(st1row_ref[0, 0:1, :], ws1_ref[2 * H2:3 * H2, :])
    t = _relu(_mm(pool, ws1_ref[0:H2, :]) + _mm(s1b, ws1_ref[H2:2 * H2, :])
              + sconst + bs1_ref[...]).astype(jnp.bfloat16)
    t = _relu(_mm(t, ws2_ref[...]) + bs2_ref[...]).astype(jnp.bfloat16)
    s2out = _relu(_mm(t, ws3_ref[...]) + bs3_ref[...])
    osites_ref[0] = sites_ref[0] + s2out

    mean_row = jnp.sum(s2out, axis=0, keepdims=True) / float(N)
    rows = lax.broadcasted_iota(jnp.int32, (8, H2), 0)
    contrib = jnp.where(rows == b, jnp.broadcast_to(mean_row, (8, H2)), 0.0)

    @pl.when(b == 0)
    def _init():
        smean_ref[...] = contrib

    @pl.when(b != 0)
    def _acc():
        smean_ref[...] = smean_ref[...] + contrib

    @pl.when(b == B - 1)
    def _states():
        bmean = (esum_ref[...] / float(E)).astype(jnp.bfloat16)
        v = (_mm(bmean, wt1_ref[0:H2, :])
             + _mm(smean_ref[...].astype(jnp.bfloat16), wt1_ref[H2:2 * H2, :])
             + _mm(st1_ref[...], wt1_ref[2 * H2:3 * H2, :]) + bt1_ref[...])
        v = _relu(v).astype(jnp.bfloat16)
        v = _relu(_mm(v, wt2_ref[...]) + bt2_ref[...]).astype(jnp.bfloat16)
        v = _relu(_mm(v, wt3_ref[...]) + bt3_ref[...])
        ostates_ref[...] = stpad_ref[...] + v


def _run_site(poola, poolb, sites1, sites, states1_pad, states_pad, esum, params):
    (ws1, bs1), (ws2, bs2), (ws3, bs3) = params['site_mlp']
    (wt1, bt1), (wt2, bt2), (wt3, bt3) = params['state_mlp']
    pool5 = poola.reshape(NC, N, B + 1, 1, D)
    pool5b = poolb.reshape(NC, N, B + 1, 1, D)
    st1rows = states1_pad.reshape(8, 1, D)
    return pl.pallas_call(
        _site_body,
        grid=(B,),
        in_specs=[
            pl.BlockSpec((NC, N, 1, 1, D), lambda b: (0, 0, b, 0, 0)),
            pl.BlockSpec((NC, N, 1, 1, D), lambda b: (0, 0, b, 0, 0)),
            pl.BlockSpec((NC, N, 1, 1, D), lambda b: (0, 0, B, 0, 0)),
            pl.BlockSpec((NC, N, 1, 1, D), lambda b: (0, 0, B, 0, 0)),
            pl.BlockSpec((1, N, D), lambda b: (b, 0, 0)),
            pl.BlockSpec((1, N, D), lambda b: (b, 0, 0)),
            pl.BlockSpec((1, 1, D), lambda b: (b, 0, 0)),
            pl.BlockSpec((8, D), lambda b: (0, 0)),
            pl.BlockSpec((8, D), lambda b: (0, 0)),
            pl.BlockSpec((8, H2), lambda b: (0, 0)),
            pl.BlockSpec((3 * H2, 256), lambda b: (0, 0)),
            pl.BlockSpec((1, 256), lambda b: (0, 0)),
            pl.BlockSpec((256, 256), lambda b: (0, 0)),
            pl.BlockSpec((1, 256), lambda b: (0, 0)),
            pl.BlockSpec((256, H2), lambda b: (0, 0)),
            pl.BlockSpec((1, H2), lambda b: (0, 0)),
            pl.BlockSpec((3 * H2, 256), lambda b: (0, 0)),
            pl.BlockSpec((1, 256), lambda b: (0, 0)),
            pl.BlockSpec((256, 256), lambda b: (0, 0)),
            pl.BlockSpec((1, 256), lambda b: (0, 0)),
            pl.BlockSpec((256, H2), lambda b: (0, 0)),
            pl.BlockSpec((1, H2), lambda b: (0, 0)),
        ],
        out_specs=[
            pl.BlockSpec((1, N, D), lambda b: (b, 0, 0)),
            pl.BlockSpec((8, D), lambda b: (0, 0)),
        ],
        out_shape=[
            jax.ShapeDtypeStruct((B, N, D), jnp.float32),
            jax.ShapeDtypeStruct((8, D), jnp.float32),
        ],
        scratch_shapes=[pltpu.VMEM((8, H2), jnp.float32)],
    )(pool5, pool5b, pool5, pool5b, sites1, sites, st1rows, states1_pad, states_pad, esum,
      ws1.astype(jnp.bfloat16), bs1.reshape(1, 256),
      ws2.astype(jnp.bfloat16), bs2.reshape(1, 256),
      ws3.astype(jnp.bfloat16), bs3.reshape(1, H2),
      wt1.astype(jnp.bfloat16), bt1.reshape(1, 256),
      wt2.astype(jnp.bfloat16), bt2.reshape(1, 256),
      wt3.astype(jnp.bfloat16), bt3.reshape(1, H2))


def kernel(sites, bonds, states, indices1, indices2, params):
    idx1 = indices1.astype(jnp.int32)
    idx2 = indices2.astype(jnp.int32)
    states_pad = jnp.pad(states, ((0, 8 - B), (0, 0)))

    states1_pad = _run_prenet(states_pad.reshape(8, 1, D),
                              params['states_fc']).reshape(8, H2)
    table, sites1 = _run_sites_prenet(sites, params['sites_fc'])

    g = _sc_gather(table, jnp.concatenate([idx1, idx2]))
    bonds_out, b2t, esum = _run_edge(bonds, g, states1_pad, params, 0, E)
    pool = _sc_scatter(b2t.reshape(E, FS), idx1)
    sites_out, states_out_pad = _run_site(pool, pool, sites1, sites,
                                          states1_pad, states_pad,
                                          esum, params)
    return sites_out, bonds_out, states_out_pad[:B]


# speedup vs baseline: 1.7376x; 1.0073x over previous
import functools

import jax
import jax.numpy as jnp
from jax import lax
from jax.experimental import pallas as pl
from jax.experimental.pallas import tpu as pltpu
from jax.experimental.pallas import tpu_sc as plsc

B, N, E, D = 4, 2048, 32768, 128
H1, H2 = 256, 128
NC, NS = 2, 16
NW = NC * NS
GC = 128
SC_CHUNK = 32
BE = 2048
F = B * D


def _relu(x):
    return jnp.maximum(x, 0.0)


def _mm(x, w):
    return jax.lax.dot_general(x, w, (((x.ndim - 1,), (0,)), ((), ())),
                               preferred_element_type=jnp.float32)


def _prenet_body(x_ref, w1_ref, b1_ref, w2_ref, b2_ref, o_ref):
    x = x_ref[0].astype(jnp.bfloat16)
    h = _relu(_mm(x, w1_ref[...]) + b1_ref[...]).astype(jnp.bfloat16)
    o_ref[0] = _relu(_mm(h, w2_ref[...]) + b2_ref[...]).astype(jnp.bfloat16)


def _run_prenet(x, wb):
    (w1, b1), (w2, b2) = wb
    g, r, d = x.shape
    return pl.pallas_call(
        _prenet_body,
        grid=(g,),
        in_specs=[
            pl.BlockSpec((1, r, d), lambda i: (i, 0, 0)),
            pl.BlockSpec((d, H1), lambda i: (0, 0)),
            pl.BlockSpec((1, H1), lambda i: (0, 0)),
            pl.BlockSpec((H1, H2), lambda i: (0, 0)),
            pl.BlockSpec((1, H2), lambda i: (0, 0)),
        ],
        out_specs=pl.BlockSpec((1, r, H2), lambda i: (i, 0, 0)),
        out_shape=jax.ShapeDtypeStruct((g, r, H2), jnp.bfloat16),
    )(x, w1.astype(jnp.bfloat16), b1.reshape(1, H1),
      w2.astype(jnp.bfloat16), b2.reshape(1, H2))


def _sites_prenet_body(x_ref, w1_ref, b1_ref, w2_ref, b2_ref,
                       tab_ref, s1_ref):
    ys = []
    for b in range(B):
        x = x_ref[b].astype(jnp.bfloat16)
        h = _relu(_mm(x, w1_ref[...]) + b1_ref[...]).astype(jnp.bfloat16)
        y = _relu(_mm(h, w2_ref[...]) + b2_ref[...]).astype(jnp.bfloat16)
        s1_ref[b] = y
        ys.append(y)
    for b in range(2):
        hi = jax.lax.bitcast_convert_type(ys[b], jnp.uint16).astype(jnp.uint32)
        lo = jax.lax.bitcast_convert_type(ys[b + 2], jnp.uint16).astype(jnp.uint32)
        w = (hi << 16) | lo
        tab_ref[:, b * D:(b + 1) * D] = jax.lax.bitcast_convert_type(w, jnp.int32)


def _run_sites_prenet(sites, wb):
    (w1, b1), (w2, b2) = wb
    return pl.pallas_call(
        _sites_prenet_body,
        in_specs=[
            pl.BlockSpec((B, N, D), lambda: (0, 0, 0)),
            pl.BlockSpec((D, H1), lambda: (0, 0)),
            pl.BlockSpec((1, H1), lambda: (0, 0)),
            pl.BlockSpec((H1, H2), lambda: (0, 0)),
            pl.BlockSpec((1, H2), lambda: (0, 0)),
        ],
        out_specs=[
            pl.BlockSpec((N, F // 2), lambda: (0, 0)),
            pl.BlockSpec((B, N, H2), lambda: (0, 0, 0)),
        ],
        out_shape=[
            jax.ShapeDtypeStruct((N, F // 2), jnp.int32),
            jax.ShapeDtypeStruct((B, N, H2), jnp.bfloat16),
        ],
    )(sites, w1.astype(jnp.bfloat16), b1.reshape(1, H1),
      w2.astype(jnp.bfloat16), b2.reshape(1, H2))


def _sc_gather_body(nrows, table_hbm, idx_hbm, out_hbm, idx_v, rows_v, sem):
    wid = lax.axis_index("s") * NC + lax.axis_index("c")
    rows_per_w = nrows // NW
    base = wid * rows_per_w
    for k in range(rows_per_w // GC):
        start = base + k * GC
        pltpu.sync_copy(idx_hbm.at[pl.ds(start, GC)], idx_v)
        pltpu.async_copy(table_hbm.at[idx_v], rows_v, sem).wait()
        pltpu.sync_copy(rows_v, out_hbm.at[pl.ds(start, GC)])


def _sc_gather(table, idx_cat):
    nrows = idx_cat.shape[0]
    mesh = plsc.VectorSubcoreMesh(core_axis_name="c", subcore_axis_name="s",
                                  num_cores=NC, num_subcores=NS)
    fn = pl.kernel(
        functools.partial(_sc_gather_body, nrows),
        out_type=jax.ShapeDtypeStruct((nrows, F // 2), jnp.int32),
        mesh=mesh,
        scratch_types=[
            pltpu.VMEM((GC,), jnp.int32),
            pltpu.VMEM((GC, F // 2), jnp.int32),
            pltpu.SemaphoreType.DMA,
        ],
    )
    return fn(table, idx_cat)


def _edge_body(bonds_ref, s1_ref, s2_ref, st1_ref,
               wb1_ref, bb1_ref, wb2_ref, bb2_ref,
               wm1_ref, bm1_ref, wm2_ref, bm2_ref, wm3_ref, bm3_ref,
               outb_ref, b2t_ref, esum_ref):
    x_all = bonds_ref[...].reshape(B * BE, D)
    xb = x_all.astype(jnp.bfloat16)
    h = _relu(_mm(xb, wb1_ref[...]) + bb1_ref[...]).astype(jnp.bfloat16)
    bonds1 = _relu(_mm(h, wb2_ref[...]) + bb2_ref[...]).astype(jnp.bfloat16)

    mask = jnp.int32(-65536)

    def unpack(u_ref):
        u = u_ref[...]
        hi = jax.lax.bitcast_convert_type(u & mask, jnp.float32)
        lo = jax.lax.bitcast_convert_type(u << 16, jnp.float32)
        return jnp.concatenate([hi[:, 0:D], hi[:, D:2 * D],
                                lo[:, 0:D], lo[:, D:2 * D]],
                               axis=0).astype(jnp.bfloat16)

    s1_all = unpack(s1_ref)
    s2_all = unpack(s2_ref)
    t = (_mm(s1_all, wm1_ref[0:H2, :]) + _mm(s2_all, wm1_ref[H2:2 * H2, :])
         + _mm(bonds1, wm1_ref[2 * H2:3 * H2, :]))
    sconst = _mm(st1_ref[0:B, :], wm1_ref[3 * H2:4 * H2, :])
    cadd = jnp.concatenate(
        [jnp.broadcast_to(sconst[b:b + 1, :], (BE, 256)) for b in range(B)],
        axis=0)
    t = _relu(t + cadd + bm1_ref[...]).astype(jnp.bfloat16)
    t = _relu(_mm(t, wm2_ref[...]) + bm2_ref[...]).astype(jnp.bfloat16)
    b2 = _mm(t, wm3_ref[...]) + bm3_ref[...]
    outb_ref[...] = (x_all + b2).reshape(B, BE, D)
    parts = []
    for b in range(B):
        blk = b2[b * BE:(b + 1) * BE, :]
        b2t_ref[:, b, :] = blk
        parts.append(jnp.sum(blk, axis=0, keepdims=True))
    b2t_ref[:, B, :] = jnp.ones((BE, D), jnp.float32)
    parts.append(jnp.zeros((8 - B, H2), jnp.float32))
    psum = jnp.concatenate(parts, axis=0)

    @pl.when(pl.program_id(0) == 0)
    def _init():
        esum_ref[...] = psum

    @pl.when(pl.program_id(0) != 0)
    def _acc():
        esum_ref[...] = esum_ref[...] + psum


def _run_edge(bonds, g, states1_pad, params, off, ne):
    (wb1, bb1), (wb2, bb2) = params['bonds_fc']
    (wm1, bm1), (wm2, bm2), (wm3, bm3) = params['bond_mlp']
    nblk = ne // BE
    oblk = off // BE
    return pl.pallas_call(
        _edge_body,
        grid=(nblk,),
        in_specs=[
            pl.BlockSpec((B, BE, D), lambda e: (0, e + oblk, 0)),
            pl.BlockSpec((BE, F // 2), lambda e: (e, 0)),
            pl.BlockSpec((BE, F // 2), lambda e: (e + nblk, 0)),
            pl.BlockSpec((8, D), lambda e: (0, 0)),
            pl.BlockSpec((D, H1), lambda e: (0, 0)),
            pl.BlockSpec((1, H1), lambda e: (0, 0)),
            pl.BlockSpec((H1, H2), lambda e: (0, 0)),
            pl.BlockSpec((1, H2), lambda e: (0, 0)),
            pl.BlockSpec((4 * H2, 256), lambda e: (0, 0)),
            pl.BlockSpec((1, 256), lambda e: (0, 0)),
            pl.BlockSpec((256, 256), lambda e: (0, 0)),
            pl.BlockSpec((1, 256), lambda e: (0, 0)),
            pl.BlockSpec((256, H2), lambda e: (0, 0)),
            pl.BlockSpec((1, H2), lambda e: (0, 0)),
        ],
        out_specs=[
            pl.BlockSpec((B, BE, D), lambda e: (0, e, 0)),
            pl.BlockSpec((BE, B + 1, D), lambda e: (e, 0, 0)),
            pl.BlockSpec((8, H2), lambda e: (0, 0)),
        ],
        out_shape=[
            jax.ShapeDtypeStruct((B, ne, D), jnp.float32),
            jax.ShapeDtypeStruct((ne, B + 1, D), jnp.float32),
            jax.ShapeDtypeStruct((8, H2), jnp.float32),
        ],
    )(bonds, g, g, states1_pad,
      wb1.astype(jnp.bfloat16), bb1.reshape(1, H1),
      wb2.astype(jnp.bfloat16), bb2.reshape(1, H2),
      wm1.astype(jnp.bfloat16), bm1.reshape(1, 256),
      wm2.astype(jnp.bfloat16), bm2.reshape(1, 256),
      wm3.astype(jnp.bfloat16), bm3.reshape(1, H2))


FS = (B + 1) * D


def _sc_scatter_body(ne, b2t_hbm, idx_hbm, zrow_hbm, pool_hbm,
                     rows_v, idx_v, lsem, isem, ssem):
    cid = lax.axis_index("c")
    sid = lax.axis_index("s")
    wid = sid * NC + cid
    zrows = N // NS
    r0 = sid * zrows
    pltpu.sync_copy(zrow_hbm, rows_v.at[0])
    zds = [pltpu.async_copy(rows_v.at[0],
                            pool_hbm.at[cid, pl.ds(r0 + t * 32, 32)], lsem)
           for t in range(zrows // 32)]
    for d in zds:
        d.wait()
    plsc.subcore_barrier()
    pool_c = pool_hbm.at[cid]
    e_per_w = ne // NW
    base = wid * e_per_w
    nch = e_per_w // SC_CHUNK
    nb = 4
    loads = [None] * nb
    scats = [[] for _ in range(nb)]

    def start_load(k):
        bi = k % nb
        st = base + k * SC_CHUNK
        loads[bi] = (
            pltpu.async_copy(b2t_hbm.at[pl.ds(st, SC_CHUNK)], rows_v.at[bi],
                             lsem),
            pltpu.async_copy(idx_hbm.at[pl.ds(st, SC_CHUNK)], idx_v.at[bi],
                             isem),
        )

    start_load(0)
    start_load(1)
    for k in range(nch):
        bi = k % nb
        for d in loads[bi]:
            d.wait()
        for j in range(SC_CHUNK // 16):
            idx_vec = idx_v[bi, pl.ds(j * 16, 16)]
            scats[bi].append(
                pltpu.async_copy(rows_v.at[bi, pl.ds(j * 16, 16)],
                                 pool_c.at[idx_vec], ssem.at[bi], add=True))
        if k + 2 < nch:
            nbi = (k + 2) % nb
            for d in scats[nbi]:
                d.wait()
            scats[nbi] = []
            start_load(k + 2)
    for bl in scats:
        for d in bl:
            d.wait()


def _sc_scatter(b2t, idx1):
    ne = idx1.shape[0]
    mesh = plsc.VectorSubcoreMesh(core_axis_name="c", subcore_axis_name="s",
                                  num_cores=NC, num_subcores=NS)
    fn = pl.kernel(
        functools.partial(_sc_scatter_body, ne),
        out_type=jax.ShapeDtypeStruct((NC, N, FS), jnp.float32),
        mesh=mesh,
        scratch_types=[
            pltpu.VMEM((4, SC_CHUNK, FS), jnp.float32),
            pltpu.VMEM((4, SC_CHUNK), jnp.int32),
            pltpu.SemaphoreType.DMA,
            pltpu.SemaphoreType.DMA,
            pltpu.SemaphoreType.DMA((4,)),
        ],
    )
    zrow = jnp.zeros((32, FS), jnp.float32)
    return fn(b2t, idx1, zrow)


def _site_body(pool_ref, poolb_ref, cnt_ref, cntb_ref, sites1_ref, sites_ref, st1row_ref,
               st1_ref, stpad_ref, esum_ref,
               ws1_ref, bs1_ref, ws2_ref, bs2_ref, ws3_ref, bs3_ref,
               wt1_ref, bt1_ref, wt2_ref, bt2_ref, wt3_ref, bt3_ref,
               osites_ref, ostates_ref, smean_ref):
    b = pl.program_id(0)
    psum = pool_ref[0, :, 0, 0, :] + pool_ref[1, :, 0, 0, :]
    c = cnt_ref[0, :, 0, 0, 0:1] + cnt_ref[1, :, 0, 0, 0:1]
    pool = (psum / jnp.maximum(c, 1.0)).astype(jnp.bfloat16)
    s1b = sites1_ref[0]
    sconst = _mm(st1row_ref[0, 0:1, :], ws1_ref[2 * H2:3 * H2, :])
    t = _relu(_mm(pool, ws1_ref[0:H2, :]) + _mm(s1b, ws1_ref[H2:2 * H2, :])
              + sconst + bs1_ref[...]).astype(jnp.bfloat16)
    t = _relu(_mm(t, ws2_ref[...]) + bs2_ref[...]).astype(jnp.bfloat16)
    s2out = _relu(_mm(t, ws3_ref[...]) + bs3_ref[...])
    osites_ref[0] = sites_ref[0] + s2out

    mean_row = jnp.sum(s2out, axis=0, keepdims=True) / float(N)
    rows = lax.broadcasted_iota(jnp.int32, (8, H2), 0)
    contrib = jnp.where(rows == b, jnp.broadcast_to(mean_row, (8, H2)), 0.0)

    @pl.when(b == 0)
    def _init():
        smean_ref[...] = contrib

    @pl.when(b != 0)
    def _acc():
        smean_ref[...] = smean_ref[...] + contrib

    @pl.when(b == B - 1)
    def _states():
        bmean = (esum_ref[...] / float(E)).astype(jnp.bfloat16)
        v = (_mm(bmean, wt1_ref[0:H2, :])
             + _mm(smean_ref[...].astype(jnp.bfloat16), wt1_ref[H2:2 * H2, :])
             + _mm(st1_ref[...], wt1_ref[2 * H2:3 * H2, :]) + bt1_ref[...])
        v = _relu(v).astype(jnp.bfloat16)
        v = _relu(_mm(v, wt2_ref[...]) + bt2_ref[...]).astype(jnp.bfloat16)
        v = _relu(_mm(v, wt3_ref[...]) + bt3_ref[...])
        ostates_ref[...] = stpad_ref[...] + v


def _run_site(poola, poolb, sites1, sites, states1_pad, states_pad, esum, params):
    (ws1, bs1), (ws2, bs2), (ws3, bs3) = params['site_mlp']
    (wt1, bt1), (wt2, bt2), (wt3, bt3) = params['state_mlp']
    pool5 = poola.reshape(NC, N, B + 1, 1, D)
    pool5b = poolb.reshape(NC, N, B + 1, 1, D)
    st1rows = states1_pad.reshape(8, 1, D)
    return pl.pallas_call(
        _site_body,
        grid=(B,),
        in_specs=[
            pl.BlockSpec((NC, N, 1, 1, D), lambda b: (0, 0, b, 0, 0)),
            pl.BlockSpec((NC, N, 1, 1, D), lambda b: (0, 0, b, 0, 0)),
            pl.BlockSpec((NC, N, 1, 1, D), lambda b: (0, 0, B, 0, 0)),
            pl.BlockSpec((NC, N, 1, 1, D), lambda b: (0, 0, B, 0, 0)),
            pl.BlockSpec((1, N, D), lambda b: (b, 0, 0)),
            pl.BlockSpec((1, N, D), lambda b: (b, 0, 0)),
            pl.BlockSpec((1, 1, D), lambda b: (b, 0, 0)),
            pl.BlockSpec((8, D), lambda b: (0, 0)),
            pl.BlockSpec((8, D), lambda b: (0, 0)),
            pl.BlockSpec((8, H2), lambda b: (0, 0)),
            pl.BlockSpec((3 * H2, 256), lambda b: (0, 0)),
            pl.BlockSpec((1, 256), lambda b: (0, 0)),
            pl.BlockSpec((256, 256), lambda b: (0, 0)),
            pl.BlockSpec((1, 256), lambda b: (0, 0)),
            pl.BlockSpec((256, H2), lambda b: (0, 0)),
            pl.BlockSpec((1, H2), lambda b: (0, 0)),
            pl.BlockSpec((3 * H2, 256), lambda b: (0, 0)),
            pl.BlockSpec((1, 256), lambda b: (0, 0)),
            pl.BlockSpec((256, 256), lambda b: (0, 0)),
            pl.BlockSpec((1, 256), lambda b: (0, 0)),
            pl.BlockSpec((256, H2), lambda b: (0, 0)),
            pl.BlockSpec((1, H2), lambda b: (0, 0)),
        ],
        out_specs=[
            pl.BlockSpec((1, N, D), lambda b: (b, 0, 0)),
            pl.BlockSpec((8, D), lambda b: (0, 0)),
        ],
        out_shape=[
            jax.ShapeDtypeStruct((B, N, D), jnp.float32),
            jax.ShapeDtypeStruct((8, D), jnp.float32),
        ],
        scratch_shapes=[pltpu.VMEM((8, H2), jnp.float32)],
    )(pool5, pool5b, pool5, pool5b, sites1, sites, st1rows, states1_pad, states_pad, esum,
      ws1.astype(jnp.bfloat16), bs1.reshape(1, 256),
      ws2.astype(jnp.bfloat16), bs2.reshape(1, 256),
      ws3.astype(jnp.bfloat16), bs3.reshape(1, H2),
      wt1.astype(jnp.bfloat16), bt1.reshape(1, 256),
      wt2.astype(jnp.bfloat16), bt2.reshape(1, 256),
      wt3.astype(jnp.bfloat16), bt3.reshape(1, H2))


def kernel(sites, bonds, states, indices1, indices2, params):
    idx1 = indices1.astype(jnp.int32)
    idx2 = indices2.astype(jnp.int32)
    states_pad = jnp.pad(states, ((0, 8 - B), (0, 0)))

    states1_pad = _run_prenet(states_pad.reshape(8, 1, D),
                              params['states_fc']).reshape(8, H2)
    table, sites1 = _run_sites_prenet(sites, params['sites_fc'])

    g = _sc_gather(table, jnp.concatenate([idx1, idx2]))
    bonds_out, b2t, esum = _run_edge(bonds, g, states1_pad, params, 0, E)
    pool = _sc_scatter(b2t.reshape(E, FS), idx1)
    sites_out, states_out_pad = _run_site(pool, pool, sites1, sites,
                                          states1_pad, states_pad,
                                          esum, params)
    return sites_out, bonds_out, states_out_pad[:B]


# speedup vs baseline: 1.7761x; 1.0221x over previous
import functools

import jax
import jax.numpy as jnp
from jax import lax
from jax.experimental import pallas as pl
from jax.experimental.pallas import tpu as pltpu
from jax.experimental.pallas import tpu_sc as plsc

B, N, E, D = 4, 2048, 32768, 128
H1, H2 = 256, 128
NC, NS = 2, 16
NW = NC * NS
GC = 128
SC_CHUNK = 32
BE = 2048
F = B * D


def _relu(x):
    return jnp.maximum(x, 0.0)


def _mm(x, w):
    return jax.lax.dot_general(x, w, (((x.ndim - 1,), (0,)), ((), ())),
                               preferred_element_type=jnp.float32)


def _prenet_body(x_ref, w1_ref, b1_ref, w2_ref, b2_ref, o_ref):
    x = x_ref[0].astype(jnp.bfloat16)
    h = _relu(_mm(x, w1_ref[...]) + b1_ref[...]).astype(jnp.bfloat16)
    o_ref[0] = _relu(_mm(h, w2_ref[...]) + b2_ref[...]).astype(jnp.bfloat16)


def _run_prenet(x, wb):
    (w1, b1), (w2, b2) = wb
    g, r, d = x.shape
    return pl.pallas_call(
        _prenet_body,
        grid=(g,),
        in_specs=[
            pl.BlockSpec((1, r, d), lambda i: (i, 0, 0)),
            pl.BlockSpec((d, H1), lambda i: (0, 0)),
            pl.BlockSpec((1, H1), lambda i: (0, 0)),
            pl.BlockSpec((H1, H2), lambda i: (0, 0)),
            pl.BlockSpec((1, H2), lambda i: (0, 0)),
        ],
        out_specs=pl.BlockSpec((1, r, H2), lambda i: (i, 0, 0)),
        out_shape=jax.ShapeDtypeStruct((g, r, H2), jnp.bfloat16),
    )(x, w1.astype(jnp.bfloat16), b1.reshape(1, H1),
      w2.astype(jnp.bfloat16), b2.reshape(1, H2))


def _sites_prenet_body(x_ref, w1_ref, b1_ref, w2_ref, b2_ref,
                       tab_ref, s1_ref):
    ys = []
    for b in range(B):
        x = x_ref[b].astype(jnp.bfloat16)
        h = _relu(_mm(x, w1_ref[...]) + b1_ref[...]).astype(jnp.bfloat16)
        y = _relu(_mm(h, w2_ref[...]) + b2_ref[...]).astype(jnp.bfloat16)
        s1_ref[b] = y
        ys.append(y)
    for b in range(2):
        hi = jax.lax.bitcast_convert_type(ys[b], jnp.uint16).astype(jnp.uint32)
        lo = jax.lax.bitcast_convert_type(ys[b + 2], jnp.uint16).astype(jnp.uint32)
        w = (hi << 16) | lo
        tab_ref[:, b * D:(b + 1) * D] = jax.lax.bitcast_convert_type(w, jnp.int32)


def _run_sites_prenet(sites, wb):
    (w1, b1), (w2, b2) = wb
    return pl.pallas_call(
        _sites_prenet_body,
        in_specs=[
            pl.BlockSpec((B, N, D), lambda: (0, 0, 0)),
            pl.BlockSpec((D, H1), lambda: (0, 0)),
            pl.BlockSpec((1, H1), lambda: (0, 0)),
            pl.BlockSpec((H1, H2), lambda: (0, 0)),
            pl.BlockSpec((1, H2), lambda: (0, 0)),
        ],
        out_specs=[
            pl.BlockSpec((N, F // 2), lambda: (0, 0)),
            pl.BlockSpec((B, N, H2), lambda: (0, 0, 0)),
        ],
        out_shape=[
            jax.ShapeDtypeStruct((N, F // 2), jnp.int32),
            jax.ShapeDtypeStruct((B, N, H2), jnp.bfloat16),
        ],
    )(sites, w1.astype(jnp.bfloat16), b1.reshape(1, H1),
      w2.astype(jnp.bfloat16), b2.reshape(1, H2))


def _sc_gather_body(nrows, table_hbm, idx_hbm, out_hbm, idx_v, rows_v,
                    isem, gsem, ssem):
    wid = lax.axis_index("s") * NC + lax.axis_index("c")
    rows_per_w = nrows // NW
    base = wid * rows_per_w
    nch = rows_per_w // GC
    i_desc = [None] * 4
    g_desc = [None] * 3
    s_desc = [None] * 3

    def load_idx(m):
        bi = m % 4
        i_desc[bi] = pltpu.async_copy(
            idx_hbm.at[pl.ds(base + m * GC, GC)], idx_v.at[bi], isem.at[bi])

    def fire_gather(m):
        bi = m % 3
        if m >= 3:
            s_desc[bi].wait()
        i_desc[m % 4].wait()
        g_desc[bi] = pltpu.async_copy(
            table_hbm.at[idx_v.at[m % 4]], rows_v.at[bi], gsem.at[bi])

    def fire_store(m):
        bi = m % 3
        g_desc[bi].wait()
        s_desc[bi] = pltpu.async_copy(
            rows_v.at[bi], out_hbm.at[pl.ds(base + m * GC, GC)], ssem.at[bi])

    for m in range(min(3, nch)):
        load_idx(m)
    fire_gather(0)
    for k in range(nch):
        if k + 3 < nch:
            load_idx(k + 3)
        if k + 1 < nch:
            fire_gather(k + 1)
        fire_store(k)
    for d in s_desc:
        if d is not None:
            d.wait()


def _sc_gather(table, idx_cat):
    nrows = idx_cat.shape[0]
    mesh = plsc.VectorSubcoreMesh(core_axis_name="c", subcore_axis_name="s",
                                  num_cores=NC, num_subcores=NS)
    fn = pl.kernel(
        functools.partial(_sc_gather_body, nrows),
        out_type=jax.ShapeDtypeStruct((nrows, F // 2), jnp.int32),
        mesh=mesh,
        scratch_types=[
            pltpu.VMEM((4, GC), jnp.int32),
            pltpu.VMEM((3, GC, F // 2), jnp.int32),
            pltpu.SemaphoreType.DMA((4,)),
            pltpu.SemaphoreType.DMA((3,)),
            pltpu.SemaphoreType.DMA((3,)),
        ],
    )
    return fn(table, idx_cat)


def _edge_body(bonds_ref, s1_ref, s2_ref, st1_ref,
               wb1_ref, bb1_ref, wb2_ref, bb2_ref,
               wm1_ref, bm1_ref, wm2_ref, bm2_ref, wm3_ref, bm3_ref,
               outb_ref, b2t_ref, esum_ref):
    x_all = bonds_ref[...].reshape(B * BE, D)
    xb = x_all.astype(jnp.bfloat16)
    h = _relu(_mm(xb, wb1_ref[...]) + bb1_ref[...]).astype(jnp.bfloat16)
    bonds1 = _relu(_mm(h, wb2_ref[...]) + bb2_ref[...]).astype(jnp.bfloat16)

    mask = jnp.int32(-65536)

    def unpack(u_ref):
        u = u_ref[...]
        hi = jax.lax.bitcast_convert_type(u & mask, jnp.float32)
        lo = jax.lax.bitcast_convert_type(u << 16, jnp.float32)
        return jnp.concatenate([hi[:, 0:D], hi[:, D:2 * D],
                                lo[:, 0:D], lo[:, D:2 * D]],
                               axis=0).astype(jnp.bfloat16)

    s1_all = unpack(s1_ref)
    s2_all = unpack(s2_ref)
    t = (_mm(s1_all, wm1_ref[0:H2, :]) + _mm(s2_all, wm1_ref[H2:2 * H2, :])
         + _mm(bonds1, wm1_ref[2 * H2:3 * H2, :]))
    sconst = _mm(st1_ref[0:B, :], wm1_ref[3 * H2:4 * H2, :])
    cadd = jnp.concatenate(
        [jnp.broadcast_to(sconst[b:b + 1, :], (BE, 256)) for b in range(B)],
        axis=0)
    t = _relu(t + cadd + bm1_ref[...]).astype(jnp.bfloat16)
    t = _relu(_mm(t, wm2_ref[...]) + bm2_ref[...]).astype(jnp.bfloat16)
    b2 = _mm(t, wm3_ref[...]) + bm3_ref[...]
    outb_ref[...] = (x_all + b2).reshape(B, BE, D)
    parts = []
    for b in range(B):
        blk = b2[b * BE:(b + 1) * BE, :]
        b2t_ref[:, b, :] = blk
        parts.append(jnp.sum(blk, axis=0, keepdims=True))
    b2t_ref[:, B, :] = jnp.ones((BE, D), jnp.float32)
    parts.append(jnp.zeros((8 - B, H2), jnp.float32))
    psum = jnp.concatenate(parts, axis=0)

    @pl.when(pl.program_id(0) == 0)
    def _init():
        esum_ref[...] = psum

    @pl.when(pl.program_id(0) != 0)
    def _acc():
        esum_ref[...] = esum_ref[...] + psum


def _run_edge(bonds, g, states1_pad, params, off, ne):
    (wb1, bb1), (wb2, bb2) = params['bonds_fc']
    (wm1, bm1), (wm2, bm2), (wm3, bm3) = params['bond_mlp']
    nblk = ne // BE
    oblk = off // BE
    return pl.pallas_call(
        _edge_body,
        grid=(nblk,),
        in_specs=[
            pl.BlockSpec((B, BE, D), lambda e: (0, e + oblk, 0)),
            pl.BlockSpec((BE, F // 2), lambda e: (e, 0)),
            pl.BlockSpec((BE, F // 2), lambda e: (e + nblk, 0)),
            pl.BlockSpec((8, D), lambda e: (0, 0)),
            pl.BlockSpec((D, H1), lambda e: (0, 0)),
            pl.BlockSpec((1, H1), lambda e: (0, 0)),
            pl.BlockSpec((H1, H2), lambda e: (0, 0)),
            pl.BlockSpec((1, H2), lambda e: (0, 0)),
            pl.BlockSpec((4 * H2, 256), lambda e: (0, 0)),
            pl.BlockSpec((1, 256), lambda e: (0, 0)),
            pl.BlockSpec((256, 256), lambda e: (0, 0)),
            pl.BlockSpec((1, 256), lambda e: (0, 0)),
            pl.BlockSpec((256, H2), lambda e: (0, 0)),
            pl.BlockSpec((1, H2), lambda e: (0, 0)),
        ],
        out_specs=[
            pl.BlockSpec((B, BE, D), lambda e: (0, e, 0)),
            pl.BlockSpec((BE, B + 1, D), lambda e: (e, 0, 0)),
            pl.BlockSpec((8, H2), lambda e: (0, 0)),
        ],
        out_shape=[
            jax.ShapeDtypeStruct((B, ne, D), jnp.float32),
            jax.ShapeDtypeStruct((ne, B + 1, D), jnp.float32),
            jax.ShapeDtypeStruct((8, H2), jnp.float32),
        ],
    )(bonds, g, g, states1_pad,
      wb1.astype(jnp.bfloat16), bb1.reshape(1, H1),
      wb2.astype(jnp.bfloat16), bb2.reshape(1, H2),
      wm1.astype(jnp.bfloat16), bm1.reshape(1, 256),
      wm2.astype(jnp.bfloat16), bm2.reshape(1, 256),
      wm3.astype(jnp.bfloat16), bm3.reshape(1, H2))


FS = (B + 1) * D


def _sc_scatter_body(ne, b2t_hbm, idx_hbm, zrow_hbm, pool_hbm,
                     rows_v, idx_v, lsem, isem, ssem):
    cid = lax.axis_index("c")
    sid = lax.axis_index("s")
    wid = sid * NC + cid
    zrows = N // NS
    r0 = sid * zrows
    pltpu.sync_copy(zrow_hbm, rows_v.at[0])
    zds = [pltpu.async_copy(rows_v.at[0],
                            pool_hbm.at[cid, pl.ds(r0 + t * 32, 32)], lsem)
           for t in range(zrows // 32)]
    for d in zds:
        d.wait()
    plsc.subcore_barrier()
    pool_c = pool_hbm.at[cid]
    e_per_w = ne // NW
    base = wid * e_per_w
    nch = e_per_w // SC_CHUNK
    nb = 4
    loads = [None] * nb
    scats = [[] for _ in range(nb)]

    def start_load(k):
        bi = k % nb
        st = base + k * SC_CHUNK
        loads[bi] = (
            pltpu.async_copy(b2t_hbm.at[pl.ds(st, SC_CHUNK)], rows_v.at[bi],
                             lsem),
            pltpu.async_copy(idx_hbm.at[pl.ds(st, SC_CHUNK)], idx_v.at[bi],
                             isem),
        )

    start_load(0)
    start_load(1)
    for k in range(nch):
        bi = k % nb
        for d in loads[bi]:
            d.wait()
        for j in range(SC_CHUNK // 16):
            idx_vec = idx_v[bi, pl.ds(j * 16, 16)]
            scats[bi].append(
                pltpu.async_copy(rows_v.at[bi, pl.ds(j * 16, 16)],
                                 pool_c.at[idx_vec], ssem.at[bi], add=True))
        if k + 2 < nch:
            nbi = (k + 2) % nb
            for d in scats[nbi]:
                d.wait()
            scats[nbi] = []
            start_load(k + 2)
    for bl in scats:
        for d in bl:
            d.wait()


def _sc_scatter(b2t, idx1):
    ne = idx1.shape[0]
    mesh = plsc.VectorSubcoreMesh(core_axis_name="c", subcore_axis_name="s",
                                  num_cores=NC, num_subcores=NS)
    fn = pl.kernel(
        functools.partial(_sc_scatter_body, ne),
        out_type=jax.ShapeDtypeStruct((NC, N, FS), jnp.float32),
        mesh=mesh,
        scratch_types=[
            pltpu.VMEM((4, SC_CHUNK, FS), jnp.float32),
            pltpu.VMEM((4, SC_CHUNK), jnp.int32),
            pltpu.SemaphoreType.DMA,
            pltpu.SemaphoreType.DMA,
            pltpu.SemaphoreType.DMA((4,)),
        ],
    )
    zrow = jnp.zeros((32, FS), jnp.float32)
    return fn(b2t, idx1, zrow)


def _site_body(pool_ref, poolb_ref, cnt_ref, cntb_ref, sites1_ref, sites_ref, st1row_ref,
               st1_ref, stpad_ref, esum_ref,
               ws1_ref, bs1_ref, ws2_ref, bs2_ref, ws3_ref, bs3_ref,
               wt1_ref, bt1_ref, wt2_ref, bt2_ref, wt3_ref, bt3_ref,
               osites_ref, ostates_ref, smean_ref):
    b = pl.program_id(0)
    psum = pool_ref[0, :, 0, 0, :] + pool_ref[1, :, 0, 0, :]
    c = cnt_ref[0, :, 0, 0, 0:1] + cnt_ref[1, :, 0, 0, 0:1]
    pool = (psum / jnp.maximum(c, 1.0)).astype(jnp.bfloat16)
    s1b = sites1_ref[0]
    sconst = _mm(st1row_ref[0, 0:1, :], ws1_ref[2 * H2:3 * H2, :])
    t = _relu(_mm(pool, ws1_ref[0:H2, :]) + _mm(s1b, ws1_ref[H2:2 * H2, :])
              + sconst + bs1_ref[...]).astype(jnp.bfloat16)
    t = _relu(_mm(t, ws2_ref[...]) + bs2_ref[...]).astype(jnp.bfloat16)
    s2out = _relu(_mm(t, ws3_ref[...]) + bs3_ref[...])
    osites_ref[0] = sites_ref[0] + s2out

    mean_row = jnp.sum(s2out, axis=0, keepdims=True) / float(N)
    rows = lax.broadcasted_iota(jnp.int32, (8, H2), 0)
    contrib = jnp.where(rows == b, jnp.broadcast_to(mean_row, (8, H2)), 0.0)

    @pl.when(b == 0)
    def _init():
        smean_ref[...] = contrib

    @pl.when(b != 0)
    def _acc():
        smean_ref[...] = smean_ref[...] + contrib

    @pl.when(b == B - 1)
    def _states():
        bmean = (esum_ref[...] / float(E)).astype(jnp.bfloat16)
        v = (_mm(bmean, wt1_ref[0:H2, :])
             + _mm(smean_ref[...].astype(jnp.bfloat16), wt1_ref[H2:2 * H2, :])
             + _mm(st1_ref[...], wt1_ref[2 * H2:3 * H2, :]) + bt1_ref[...])
        v = _relu(v).astype(jnp.bfloat16)
        v = _relu(_mm(v, wt2_ref[...]) + bt2_ref[...]).astype(jnp.bfloat16)
        v = _relu(_mm(v, wt3_ref[...]) + bt3_ref[...])
        ostates_ref[...] = stpad_ref[...] + v


def _run_site(poola, poolb, sites1, sites, states1_pad, states_pad, esum, params):
    (ws1, bs1), (ws2, bs2), (ws3, bs3) = params['site_mlp']
    (wt1, bt1), (wt2, bt2), (wt3, bt3) = params['state_mlp']
    pool5 = poola.reshape(NC, N, B + 1, 1, D)
    pool5b = poolb.reshape(NC, N, B + 1, 1, D)
    st1rows = states1_pad.reshape(8, 1, D)
    return pl.pallas_call(
        _site_body,
        grid=(B,),
        in_specs=[
            pl.BlockSpec((NC, N, 1, 1, D), lambda b: (0, 0, b, 0, 0)),
            pl.BlockSpec((NC, N, 1, 1, D), lambda b: (0, 0, b, 0, 0)),
            pl.BlockSpec((NC, N, 1, 1, D), lambda b: (0, 0, B, 0, 0)),
            pl.BlockSpec((NC, N, 1, 1, D), lambda b: (0, 0, B, 0, 0)),
            pl.BlockSpec((1, N, D), lambda b: (b, 0, 0)),
            pl.BlockSpec((1, N, D), lambda b: (b, 0, 0)),
            pl.BlockSpec((1, 1, D), lambda b: (b, 0, 0)),
            pl.BlockSpec((8, D), lambda b: (0, 0)),
            pl.BlockSpec((8, D), lambda b: (0, 0)),
            pl.BlockSpec((8, H2), lambda b: (0, 0)),
            pl.BlockSpec((3 * H2, 256), lambda b: (0, 0)),
            pl.BlockSpec((1, 256), lambda b: (0, 0)),
            pl.BlockSpec((256, 256), lambda b: (0, 0)),
            pl.BlockSpec((1, 256), lambda b: (0, 0)),
            pl.BlockSpec((256, H2), lambda b: (0, 0)),
            pl.BlockSpec((1, H2), lambda b: (0, 0)),
            pl.BlockSpec((3 * H2, 256), lambda b: (0, 0)),
            pl.BlockSpec((1, 256), lambda b: (0, 0)),
            pl.BlockSpec((256, 256), lambda b: (0, 0)),
            pl.BlockSpec((1, 256), lambda b: (0, 0)),
            pl.BlockSpec((256, H2), lambda b: (0, 0)),
            pl.BlockSpec((1, H2), lambda b: (0, 0)),
        ],
        out_specs=[
            pl.BlockSpec((1, N, D), lambda b: (b, 0, 0)),
            pl.BlockSpec((8, D), lambda b: (0, 0)),
        ],
        out_shape=[
            jax.ShapeDtypeStruct((B, N, D), jnp.float32),
            jax.ShapeDtypeStruct((8, D), jnp.float32),
        ],
        scratch_shapes=[pltpu.VMEM((8, H2), jnp.float32)],
    )(pool5, pool5b, pool5, pool5b, sites1, sites, st1rows, states1_pad, states_pad, esum,
      ws1.astype(jnp.bfloat16), bs1.reshape(1, 256),
      ws2.astype(jnp.bfloat16), bs2.reshape(1, 256),
      ws3.astype(jnp.bfloat16), bs3.reshape(1, H2),
      wt1.astype(jnp.bfloat16), bt1.reshape(1, 256),
      wt2.astype(jnp.bfloat16), bt2.reshape(1, 256),
      wt3.astype(jnp.bfloat16), bt3.reshape(1, H2))


def kernel(sites, bonds, states, indices1, indices2, params):
    idx1 = indices1.astype(jnp.int32)
    idx2 = indices2.astype(jnp.int32)
    states_pad = jnp.pad(states, ((0, 8 - B), (0, 0)))

    states1_pad = _run_prenet(states_pad.reshape(8, 1, D),
                              params['states_fc']).reshape(8, H2)
    table, sites1 = _run_sites_prenet(sites, params['sites_fc'])

    g = _sc_gather(table, jnp.concatenate([idx1, idx2]))
    bonds_out, b2t, esum = _run_edge(bonds, g, states1_pad, params, 0, E)
    pool = _sc_scatter(b2t.reshape(E, FS), idx1)
    sites_out, states_out_pad = _run_site(pool, pool, sites1, sites,
                                          states1_pad, states_pad,
                                          esum, params)
    return sites_out, bonds_out, states_out_pad[:B]
